# baseline (speedup 1.0000x reference)
import functools

import jax
import jax.numpy as jnp
from jax import lax
from jax.experimental import pallas as pl
from jax.experimental.pallas import tpu as pltpu
from jax.experimental.pallas import tpu_sc as plsc

E = 8
TOP_K = 2
T = 2048
D = 1024
FF = 512
NEG = -1.0e30

BT = 256
NA = T * TOP_K
PN = NA + E * BT
NB = PN // BT
NW = 32
APW = NA // NW
TPW = T // NW


def _silu(v):
    return v / (1.0 + jnp.exp(-v))


def _gate_body(x_ref, gw_ref, bias_ref, i1_ref, i2_ref, w1_ref, w2_ref):
    logits = jax.lax.dot_general(x_ref[...], gw_ref[...], (((1,), (1,)), ((), ())),
                                 preferred_element_type=jnp.float32)
    logits = logits + bias_ref[...]
    iota = jax.lax.broadcasted_iota(jnp.int32, logits.shape, 1)
    m1 = jnp.max(logits, axis=1, keepdims=True)
    i1 = jnp.min(jnp.where(logits == m1, iota, E), axis=1, keepdims=True)
    masked = jnp.where(iota == i1, NEG, logits)
    m2 = jnp.max(masked, axis=1, keepdims=True)
    i2 = jnp.min(jnp.where(masked == m2, iota, E), axis=1, keepdims=True)
    e2 = jnp.exp(m2 - m1)
    w1 = 1.0 / (1.0 + e2)
    i1_ref[...] = i1
    i2_ref[...] = i2
    w1_ref[...] = w1
    w2_ref[...] = 1.0 - w1


def _gate(x, gate_weight, bias2):
    BTG = 256
    return pl.pallas_call(
        _gate_body,
        grid=(T // BTG,),
        in_specs=[
            pl.BlockSpec((BTG, D), lambda i: (i, 0)),
            pl.BlockSpec((E, D), lambda i: (0, 0)),
            pl.BlockSpec((1, E), lambda i: (0, 0)),
        ],
        out_specs=[
            pl.BlockSpec((BTG, 1), lambda i: (i, 0)),
            pl.BlockSpec((BTG, 1), lambda i: (i, 0)),
            pl.BlockSpec((BTG, 1), lambda i: (i, 0)),
            pl.BlockSpec((BTG, 1), lambda i: (i, 0)),
        ],
        out_shape=[
            jax.ShapeDtypeStruct((T, 1), jnp.int32),
            jax.ShapeDtypeStruct((T, 1), jnp.int32),
            jax.ShapeDtypeStruct((T, 1), jnp.float32),
            jax.ShapeDtypeStruct((T, 1), jnp.float32),
        ],
    )(x, gate_weight, bias2)


_SC_MESH = plsc.VectorSubcoreMesh(core_axis_name="c", subcore_axis_name="s",
                                  num_cores=2, num_subcores=16)


def _wid():
    return lax.axis_index("s") * 2 + lax.axis_index("c")


def _sc_counts_body(ev_hbm, cnt_hbm, ev_v, cnt_v):
    wid = _wid()
    iota16 = lax.iota(jnp.int32, 16)
    pltpu.sync_copy(ev_hbm.at[pl.ds(wid * APW, APW)], ev_v)
    cnt = jnp.zeros((16,), jnp.int32)
    for c in range(APW // 16):
        ch = ev_v[pl.ds(c * 16, 16)]
        for e in range(E):
            s = jnp.sum(jnp.where(ch == e, 1, 0))
            cnt = cnt + jnp.where(iota16 == e, s, 0)
    cnt_v[...] = cnt
    pltpu.sync_copy(cnt_v, cnt_hbm.at[pl.ds(wid * 16, 16)])


_sc_counts = functools.partial(
    pl.kernel,
    out_type=jax.ShapeDtypeStruct((NW * 16,), jnp.int32),
    mesh=_SC_MESH,
    compiler_params=pltpu.CompilerParams(needs_layout_passes=False),
    scratch_types=[
        pltpu.VMEM((APW,), jnp.int32),
        pltpu.VMEM((16,), jnp.int32),
    ],
)(_sc_counts_body)


def _sc_route_body(ev_hbm, cnt_hbm, x_hbm, pos_hbm, be_hbm, xs_hbm,
                   ev_v, cnt_all_v, idx0_v, idx1_v, xrows_v, be_v, sem):
    wid = _wid()
    iota16 = lax.iota(jnp.int32, 16)
    pltpu.sync_copy(cnt_hbm, cnt_all_v)
    pltpu.sync_copy(ev_hbm.at[pl.ds(wid * APW, APW)], ev_v)

    tot = jnp.zeros((16,), jnp.int32)
    base_mine = jnp.zeros((16,), jnp.int32)
    for w in range(NW):
        row = cnt_all_v[pl.ds(w * 16, 16)]
        tot = tot + row
        base_mine = base_mine + jnp.where(w < wid, row, 0)
    padded = (tot + (BT - 1)) & (-BT)
    ends = plsc.cumsum(padded)
    off = ends - padded
    base_vec = off + base_mine

    for c in range(APW // 16):
        ch = ev_v[pl.ds(c * 16, 16)]
        poschunk = jnp.zeros((16,), jnp.int32)
        for e in range(E):
            m = ch == e
            mi = jnp.where(m, 1, 0)
            pc = plsc.cumsum(mi)
            base_e = jnp.sum(jnp.where(iota16 == e, base_vec, 0))
            poschunk = jnp.where(m, base_e + pc - 1, poschunk)
            base_vec = base_vec + jnp.where(iota16 == e, jnp.sum(mi), 0)
        if c < (APW // 32):
            idx0_v[pl.ds(c * 16, 16)] = poschunk
        else:
            idx1_v[pl.ds(c * 16 - APW // 2, 16)] = poschunk
    pltpu.sync_copy(idx0_v, pos_hbm.at[pl.ds(wid * APW, APW // 2)])
    pltpu.sync_copy(idx1_v, pos_hbm.at[pl.ds(wid * APW + APW // 2, APW // 2)])

    tok0 = (wid % 16) * APW
    half = APW // 2
    for h, idx_v in ((0, idx0_v), (1, idx1_v)):
        pltpu.sync_copy(x_hbm.at[pl.ds(tok0 + h * half, half)], xrows_v)
        pltpu.async_copy(xrows_v, xs_hbm.at[idx_v], sem).wait()

    @pl.when(wid == 0)
    def _():
        bev0 = jnp.zeros((16,), jnp.int32)
        bev1 = jnp.zeros((16,), jnp.int32)
        for e in range(E):
            end_e = jnp.sum(jnp.where(iota16 == e, ends, 0))
            bev0 = bev0 + jnp.where(iota16 * BT >= end_e, 1, 0)
            bev1 = bev1 + jnp.where((iota16 + 16) * BT >= end_e, 1, 0)
        be_v[pl.ds(0, 16)] = bev0
        be_v[pl.ds(16, 16)] = bev1
        pltpu.sync_copy(be_v, be_hbm)


_sc_route = functools.partial(
    pl.kernel,
    out_type=[
        jax.ShapeDtypeStruct((NA,), jnp.int32),
        jax.ShapeDtypeStruct((32,), jnp.int32),
        jax.ShapeDtypeStruct((PN, D), jnp.float32),
    ],
    mesh=_SC_MESH,
    compiler_params=pltpu.CompilerParams(needs_layout_passes=False),
    scratch_types=[
        pltpu.VMEM((APW,), jnp.int32),
        pltpu.VMEM((NW * 16,), jnp.int32),
        pltpu.VMEM((APW // 2,), jnp.int32),
        pltpu.VMEM((APW // 2,), jnp.int32),
        pltpu.VMEM((APW // 2, D), jnp.float32),
        pltpu.VMEM((32,), jnp.int32),
        pltpu.SemaphoreType.DMA,
    ],
)(_sc_route_body)


def _group_mlp_body(be_ref, x_ref, wg_ref, wu_ref, wd_ref, y_ref):
    e = be_ref[pl.program_id(0)]

    @pl.when(e < E)
    def _():
        x = x_ref[...]
        g = jax.lax.dot_general(x, wg_ref[0], (((1,), (1,)), ((), ())),
                                preferred_element_type=jnp.float32)
        u = jax.lax.dot_general(x, wu_ref[0], (((1,), (1,)), ((), ())),
                                preferred_element_type=jnp.float32)
        h = _silu(g) * u
        y_ref[...] = jax.lax.dot_general(h, wd_ref[0], (((1,), (1,)), ((), ())),
                                         preferred_element_type=jnp.float32)


def _group_mlp(be, xs, Wg, Wu, Wd):
    def wmap(i, s):
        return (jnp.minimum(s[i], E - 1), 0, 0)

    grid_spec = pltpu.PrefetchScalarGridSpec(
        num_scalar_prefetch=1,
        grid=(NB,),
        in_specs=[
            pl.BlockSpec((BT, D), lambda i, s: (i, 0)),
            pl.BlockSpec((1, FF, D), wmap),
            pl.BlockSpec((1, FF, D), wmap),
            pl.BlockSpec((1, D, FF), wmap),
        ],
        out_specs=pl.BlockSpec((BT, D), lambda i, s: (i, 0)),
    )
    return pl.pallas_call(
        _group_mlp_body,
        grid_spec=grid_spec,
        out_shape=jax.ShapeDtypeStruct((PN, D), jnp.float32),
    )(be, xs, Wg, Wu, Wd)


def _sc_gather_body(pos_hbm, ys_hbm, yg0_hbm, yg1_hbm, idx_v, rows_v, sem):
    wid = _wid()
    for s, out_hbm in ((0, yg0_hbm), (1, yg1_hbm)):
        pltpu.sync_copy(pos_hbm.at[pl.ds(s * T + wid * TPW, TPW)], idx_v)
        pltpu.async_copy(ys_hbm.at[idx_v], rows_v, sem).wait()
        pltpu.sync_copy(rows_v, out_hbm.at[pl.ds(wid * TPW, TPW)])


_sc_gather = functools.partial(
    pl.kernel,
    out_type=[
        jax.ShapeDtypeStruct((T, D), jnp.float32),
        jax.ShapeDtypeStruct((T, D), jnp.float32),
    ],
    mesh=_SC_MESH,
    compiler_params=pltpu.CompilerParams(needs_layout_passes=False),
    scratch_types=[
        pltpu.VMEM((TPW,), jnp.int32),
        pltpu.VMEM((TPW, D), jnp.float32),
        pltpu.SemaphoreType.DMA,
    ],
)(_sc_gather_body)


def _combine_body(x_ref, y0_ref, y1_ref, w1_ref, w2_ref,
                  wgs_ref, wus_ref, wds_ref, out_ref):
    x = x_ref[...]
    gs = jax.lax.dot_general(x, wgs_ref[...], (((1,), (1,)), ((), ())),
                             preferred_element_type=jnp.float32)
    us = jax.lax.dot_general(x, wus_ref[...], (((1,), (1,)), ((), ())),
                             preferred_element_type=jnp.float32)
    hs = _silu(gs) * us
    shared = jax.lax.dot_general(hs, wds_ref[...], (((1,), (1,)), ((), ())),
                                 preferred_element_type=jnp.float32)
    out_ref[...] = shared + w1_ref[...] * y0_ref[...] + w2_ref[...] * y1_ref[...]


def _combine(x, yg0, yg1, w1, w2, Wg_s, Wu_s, Wd_s):
    BTC = 512
    return pl.pallas_call(
        _combine_body,
        grid=(T // BTC,),
        in_specs=[
            pl.BlockSpec((BTC, D), lambda i: (i, 0)),
            pl.BlockSpec((BTC, D), lambda i: (i, 0)),
            pl.BlockSpec((BTC, D), lambda i: (i, 0)),
            pl.BlockSpec((BTC, 1), lambda i: (i, 0)),
            pl.BlockSpec((BTC, 1), lambda i: (i, 0)),
            pl.BlockSpec(Wg_s.shape, lambda i: (0, 0)),
            pl.BlockSpec(Wu_s.shape, lambda i: (0, 0)),
            pl.BlockSpec(Wd_s.shape, lambda i: (0, 0)),
        ],
        out_specs=pl.BlockSpec((BTC, D), lambda i: (i, 0)),
        out_shape=jax.ShapeDtypeStruct((T, D), jnp.float32),
    )(x, yg0, yg1, w1, w2, Wg_s, Wu_s, Wd_s)


def kernel(hidden_states, gate_weight, e_score_correction_bias, Wg, Wu, Wd,
           Wg_s, Wu_s, Wd_s):
    orig_shape = hidden_states.shape
    x = hidden_states.reshape(-1, orig_shape[-1])
    bias2 = e_score_correction_bias.reshape(1, E)

    i1, i2, w1, w2 = _gate(x, gate_weight, bias2)
    ev = jnp.concatenate([i1, i2], axis=0).reshape(NA)
    cnt = _sc_counts(ev)
    pos, be, xs = _sc_route(ev, cnt, x)
    ys = _group_mlp(be, xs, Wg, Wu, Wd)
    yg0, yg1 = _sc_gather(pos, ys)
    out = _combine(x, yg0, yg1, w1, w2, Wg_s, Wu_s, Wd_s)
    return out.reshape(orig_shape)


# speedup vs baseline: 1.0308x; 1.0308x over previous
import functools

import jax
import jax.numpy as jnp
from jax import lax
from jax.experimental import pallas as pl
from jax.experimental.pallas import tpu as pltpu
from jax.experimental.pallas import tpu_sc as plsc

E = 8
TOP_K = 2
T = 2048
D = 1024
FF = 512
NEG = -1.0e30

BT = 256
NA = T * TOP_K
PN = NA + E * BT
NB = PN // BT
NW = 32
APW = NA // NW
TPW = T // NW


def _silu(v):
    return v / (1.0 + jnp.exp(-v))


def _subcnt(idx_col):
    eq = idx_col == jax.lax.broadcasted_iota(jnp.int32, (idx_col.shape[0], 16), 1)
    return jnp.sum(jnp.where(eq, 1, 0), axis=0, keepdims=True)


def _gate_body(x_ref, gw_ref, bias_ref, i1_ref, i2_ref, w1_ref, w2_ref, cnt_ref):
    logits = jax.lax.dot_general(x_ref[...], gw_ref[...], (((1,), (1,)), ((), ())),
                                 preferred_element_type=jnp.float32)
    logits = logits + bias_ref[...]
    iota = jax.lax.broadcasted_iota(jnp.int32, logits.shape, 1)
    m1 = jnp.max(logits, axis=1, keepdims=True)
    i1 = jnp.min(jnp.where(logits == m1, iota, E), axis=1, keepdims=True)
    masked = jnp.where(iota == i1, NEG, logits)
    m2 = jnp.max(masked, axis=1, keepdims=True)
    i2 = jnp.min(jnp.where(masked == m2, iota, E), axis=1, keepdims=True)
    e2 = jnp.exp(m2 - m1)
    w1 = 1.0 / (1.0 + e2)
    i1_ref[...] = i1
    i2_ref[...] = i2
    w1_ref[...] = w1
    w2_ref[...] = 1.0 - w1
    h = jnp.concatenate([
        _subcnt(i1[0:APW, :]), _subcnt(i2[0:APW, :]),
        _subcnt(i1[APW:2 * APW, :]), _subcnt(i2[APW:2 * APW, :]),
    ], axis=0)
    cnt_ref[...] = h.reshape(2, 2, 16)


def _gate(x, gate_weight, bias2):
    BTG = 256
    return pl.pallas_call(
        _gate_body,
        grid=(T // BTG,),
        in_specs=[
            pl.BlockSpec((BTG, D), lambda i: (i, 0)),
            pl.BlockSpec((E, D), lambda i: (0, 0)),
            pl.BlockSpec((1, E), lambda i: (0, 0)),
        ],
        out_specs=[
            pl.BlockSpec((BTG, 1), lambda i: (i, 0)),
            pl.BlockSpec((BTG, 1), lambda i: (i, 0)),
            pl.BlockSpec((BTG, 1), lambda i: (i, 0)),
            pl.BlockSpec((BTG, 1), lambda i: (i, 0)),
            pl.BlockSpec((2, 2, 16), lambda i: (i, 0, 0)),
        ],
        out_shape=[
            jax.ShapeDtypeStruct((T, 1), jnp.int32),
            jax.ShapeDtypeStruct((T, 1), jnp.int32),
            jax.ShapeDtypeStruct((T, 1), jnp.float32),
            jax.ShapeDtypeStruct((T, 1), jnp.float32),
            jax.ShapeDtypeStruct((T // APW, 2, 16), jnp.int32),
        ],
    )(x, gate_weight, bias2)


_SC_MESH = plsc.VectorSubcoreMesh(core_axis_name="c", subcore_axis_name="s",
                                  num_cores=2, num_subcores=16)


def _wid():
    return lax.axis_index("s") * 2 + lax.axis_index("c")


def _sc_route_body(ev_hbm, cnt_hbm, x_hbm, pos_hbm, be_hbm, xs_hbm,
                   ev_v, cnt_all_v, idx0_v, idx1_v, xrows_v, be_v, sem):
    wid = _wid()
    iota16 = lax.iota(jnp.int32, 16)
    pltpu.sync_copy(cnt_hbm, cnt_all_v)
    pltpu.sync_copy(ev_hbm.at[pl.ds(wid * APW, APW)], ev_v)

    tot = jnp.zeros((16,), jnp.int32)
    base_mine = jnp.zeros((16,), jnp.int32)
    for w in range(NW):
        row = cnt_all_v[pl.ds((w % 16) * 32 + (w // 16) * 16, 16)]
        tot = tot + row
        base_mine = base_mine + jnp.where(w < wid, row, 0)
    padded = (tot + (BT - 1)) & (-BT)
    ends = plsc.cumsum(padded)
    off = ends - padded
    base_vec = off + base_mine

    for c in range(APW // 16):
        ch = ev_v[pl.ds(c * 16, 16)]
        poschunk = jnp.zeros((16,), jnp.int32)
        for e in range(E):
            m = ch == e
            mi = jnp.where(m, 1, 0)
            pc = plsc.cumsum(mi)
            base_e = jnp.sum(jnp.where(iota16 == e, base_vec, 0))
            poschunk = jnp.where(m, base_e + pc - 1, poschunk)
            base_vec = base_vec + jnp.where(iota16 == e, jnp.sum(mi), 0)
        if c < (APW // 32):
            idx0_v[pl.ds(c * 16, 16)] = poschunk
        else:
            idx1_v[pl.ds(c * 16 - APW // 2, 16)] = poschunk
    pltpu.sync_copy(idx0_v, pos_hbm.at[pl.ds(wid * APW, APW // 2)])
    pltpu.sync_copy(idx1_v, pos_hbm.at[pl.ds(wid * APW + APW // 2, APW // 2)])

    tok0 = (wid % 16) * APW
    half = APW // 2
    for h, idx_v in ((0, idx0_v), (1, idx1_v)):
        pltpu.sync_copy(x_hbm.at[pl.ds(tok0 + h * half, half)], xrows_v)
        pltpu.async_copy(xrows_v, xs_hbm.at[idx_v], sem).wait()

    @pl.when(wid == 0)
    def _():
        bev0 = jnp.zeros((16,), jnp.int32)
        bev1 = jnp.zeros((16,), jnp.int32)
        for e in range(E):
            end_e = jnp.sum(jnp.where(iota16 == e, ends, 0))
            bev0 = bev0 + jnp.where(iota16 * BT >= end_e, 1, 0)
            bev1 = bev1 + jnp.where((iota16 + 16) * BT >= end_e, 1, 0)
        be_v[pl.ds(0, 16)] = bev0
        be_v[pl.ds(16, 16)] = bev1
        pltpu.sync_copy(be_v, be_hbm)


_sc_route = functools.partial(
    pl.kernel,
    out_type=[
        jax.ShapeDtypeStruct((NA,), jnp.int32),
        jax.ShapeDtypeStruct((32,), jnp.int32),
        jax.ShapeDtypeStruct((PN, D), jnp.float32),
    ],
    mesh=_SC_MESH,
    compiler_params=pltpu.CompilerParams(needs_layout_passes=False),
    scratch_types=[
        pltpu.VMEM((APW,), jnp.int32),
        pltpu.VMEM((NW * 16,), jnp.int32),
        pltpu.VMEM((APW // 2,), jnp.int32),
        pltpu.VMEM((APW // 2,), jnp.int32),
        pltpu.VMEM((APW // 2, D), jnp.float32),
        pltpu.VMEM((32,), jnp.int32),
        pltpu.SemaphoreType.DMA,
    ],
)(_sc_route_body)


def _group_mlp_body(be_ref, x_ref, wg_ref, wu_ref, wd_ref, y_ref):
    e = be_ref[pl.program_id(0)]

    @pl.when(e < E)
    def _():
        x = x_ref[...]
        g = jax.lax.dot_general(x, wg_ref[0], (((1,), (1,)), ((), ())),
                                preferred_element_type=jnp.float32)
        u = jax.lax.dot_general(x, wu_ref[0], (((1,), (1,)), ((), ())),
                                preferred_element_type=jnp.float32)
        h = _silu(g) * u
        y_ref[...] = jax.lax.dot_general(h, wd_ref[0], (((1,), (1,)), ((), ())),
                                         preferred_element_type=jnp.float32)


def _group_mlp(be, xs, Wg, Wu, Wd):
    def wmap(i, s):
        return (jnp.minimum(s[i], E - 1), 0, 0)

    grid_spec = pltpu.PrefetchScalarGridSpec(
        num_scalar_prefetch=1,
        grid=(NB,),
        in_specs=[
            pl.BlockSpec((BT, D), lambda i, s: (i, 0)),
            pl.BlockSpec((1, FF, D), wmap),
            pl.BlockSpec((1, FF, D), wmap),
            pl.BlockSpec((1, D, FF), wmap),
        ],
        out_specs=pl.BlockSpec((BT, D), lambda i, s: (i, 0)),
    )
    return pl.pallas_call(
        _group_mlp_body,
        grid_spec=grid_spec,
        out_shape=jax.ShapeDtypeStruct((PN, D), jnp.float32),
    )(be, xs, Wg, Wu, Wd)


def _sc_gather_body(pos_hbm, ys_hbm, yg0_hbm, yg1_hbm, idx_v, rows_v, sem):
    wid = _wid()
    for s, out_hbm in ((0, yg0_hbm), (1, yg1_hbm)):
        pltpu.sync_copy(pos_hbm.at[pl.ds(s * T + wid * TPW, TPW)], idx_v)
        pltpu.async_copy(ys_hbm.at[idx_v], rows_v, sem).wait()
        pltpu.sync_copy(rows_v, out_hbm.at[pl.ds(wid * TPW, TPW)])


_sc_gather = functools.partial(
    pl.kernel,
    out_type=[
        jax.ShapeDtypeStruct((T, D), jnp.float32),
        jax.ShapeDtypeStruct((T, D), jnp.float32),
    ],
    mesh=_SC_MESH,
    compiler_params=pltpu.CompilerParams(needs_layout_passes=False),
    scratch_types=[
        pltpu.VMEM((TPW,), jnp.int32),
        pltpu.VMEM((TPW, D), jnp.float32),
        pltpu.SemaphoreType.DMA,
    ],
)(_sc_gather_body)


def _shared_body(x_ref, wgs_ref, wus_ref, wds_ref, out_ref):
    x = x_ref[...]
    gs = jax.lax.dot_general(x, wgs_ref[...], (((1,), (1,)), ((), ())),
                             preferred_element_type=jnp.float32)
    us = jax.lax.dot_general(x, wus_ref[...], (((1,), (1,)), ((), ())),
                             preferred_element_type=jnp.float32)
    hs = _silu(gs) * us
    out_ref[...] = jax.lax.dot_general(hs, wds_ref[...], (((1,), (1,)), ((), ())),
                                       preferred_element_type=jnp.float32)


def _shared_mlp(x, Wg_s, Wu_s, Wd_s):
    BTS = 512
    return pl.pallas_call(
        _shared_body,
        grid=(T // BTS,),
        in_specs=[
            pl.BlockSpec((BTS, D), lambda i: (i, 0)),
            pl.BlockSpec(Wg_s.shape, lambda i: (0, 0)),
            pl.BlockSpec(Wu_s.shape, lambda i: (0, 0)),
            pl.BlockSpec(Wd_s.shape, lambda i: (0, 0)),
        ],
        out_specs=pl.BlockSpec((BTS, D), lambda i: (i, 0)),
        out_shape=jax.ShapeDtypeStruct((T, D), jnp.float32),
    )(x, Wg_s, Wu_s, Wd_s)


def _combine_body(sh_ref, y0_ref, y1_ref, w1_ref, w2_ref, out_ref):
    out_ref[...] = (sh_ref[...] + w1_ref[...] * y0_ref[...]
                    + w2_ref[...] * y1_ref[...])


def _combine(shared, yg0, yg1, w1, w2):
    BTC = 512
    return pl.pallas_call(
        _combine_body,
        grid=(T // BTC,),
        in_specs=[
            pl.BlockSpec((BTC, D), lambda i: (i, 0)),
            pl.BlockSpec((BTC, D), lambda i: (i, 0)),
            pl.BlockSpec((BTC, D), lambda i: (i, 0)),
            pl.BlockSpec((BTC, 1), lambda i: (i, 0)),
            pl.BlockSpec((BTC, 1), lambda i: (i, 0)),
        ],
        out_specs=pl.BlockSpec((BTC, D), lambda i: (i, 0)),
        out_shape=jax.ShapeDtypeStruct((T, D), jnp.float32),
    )(shared, yg0, yg1, w1, w2)


def kernel(hidden_states, gate_weight, e_score_correction_bias, Wg, Wu, Wd,
           Wg_s, Wu_s, Wd_s):
    orig_shape = hidden_states.shape
    x = hidden_states.reshape(-1, orig_shape[-1])
    bias2 = e_score_correction_bias.reshape(1, E)

    i1, i2, w1, w2, cnt3 = _gate(x, gate_weight, bias2)
    ev = jnp.concatenate([i1, i2], axis=0).reshape(NA)
    cnt = cnt3.reshape(NW * 16)
    pos, be, xs = _sc_route(ev, cnt, x)
    shared = _shared_mlp(x, Wg_s, Wu_s, Wd_s)
    ys = _group_mlp(be, xs, Wg, Wu, Wd)
    yg0, yg1 = _sc_gather(pos, ys)
    out = _combine(shared, yg0, yg1, w1, w2)
    return out.reshape(orig_shape)


# speedup vs baseline: 1.1575x; 1.1229x over previous
import functools

import jax
import jax.numpy as jnp
from jax import lax
from jax.experimental import pallas as pl
from jax.experimental.pallas import tpu as pltpu
from jax.experimental.pallas import tpu_sc as plsc

E = 8
TOP_K = 2
T = 2048
D = 1024
FF = 512
NEG = -1.0e30

BT = 256
NA = T * TOP_K
PN = NA + E * BT
NB = PN // BT
NW = 32
APW = NA // NW
TPW = T // NW
DP = D // 2
MHI = -65536
RND = 32768


def _silu(v):
    return v / (1.0 + jnp.exp(-v))


def _pack_rows(v):
    b = jax.lax.bitcast_convert_type(v, jnp.int32)
    hi = (b[:, :DP] + RND) & MHI
    lo = jax.lax.shift_right_logical(b[:, DP:] + RND, 16)
    return hi | lo


def _unpack_rows(p):
    hi = jax.lax.bitcast_convert_type(p & MHI, jnp.float32)
    lo = jax.lax.bitcast_convert_type(jax.lax.shift_left(p, 16), jnp.float32)
    return jnp.concatenate([hi, lo], axis=1)


def _subcnt(idx_col):
    eq = idx_col == jax.lax.broadcasted_iota(jnp.int32, (idx_col.shape[0], 16), 1)
    return jnp.sum(jnp.where(eq, 1, 0), axis=0, keepdims=True)


def _gate_body(x_ref, gw_ref, bias_ref, i1_ref, i2_ref, w1_ref, w2_ref, cnt_ref,
               xb_ref):
    logits = jax.lax.dot_general(x_ref[...], gw_ref[...], (((1,), (1,)), ((), ())),
                                 preferred_element_type=jnp.float32)
    logits = logits + bias_ref[...]
    iota = jax.lax.broadcasted_iota(jnp.int32, logits.shape, 1)
    m1 = jnp.max(logits, axis=1, keepdims=True)
    i1 = jnp.min(jnp.where(logits == m1, iota, E), axis=1, keepdims=True)
    masked = jnp.where(iota == i1, NEG, logits)
    m2 = jnp.max(masked, axis=1, keepdims=True)
    i2 = jnp.min(jnp.where(masked == m2, iota, E), axis=1, keepdims=True)
    e2 = jnp.exp(m2 - m1)
    w1 = 1.0 / (1.0 + e2)
    i1_ref[...] = i1
    i2_ref[...] = i2
    w1_ref[...] = w1
    w2_ref[...] = 1.0 - w1
    h = jnp.concatenate([
        _subcnt(i1[0:APW, :]), _subcnt(i2[0:APW, :]),
        _subcnt(i1[APW:2 * APW, :]), _subcnt(i2[APW:2 * APW, :]),
    ], axis=0)
    cnt_ref[...] = h.reshape(2, 2, 16)
    xb_ref[...] = _pack_rows(x_ref[...])


def _gate(x, gate_weight, bias2):
    BTG = 256
    return pl.pallas_call(
        _gate_body,
        grid=(T // BTG,),
        in_specs=[
            pl.BlockSpec((BTG, D), lambda i: (i, 0)),
            pl.BlockSpec((E, D), lambda i: (0, 0)),
            pl.BlockSpec((1, E), lambda i: (0, 0)),
        ],
        out_specs=[
            pl.BlockSpec((BTG, 1), lambda i: (i, 0)),
            pl.BlockSpec((BTG, 1), lambda i: (i, 0)),
            pl.BlockSpec((BTG, 1), lambda i: (i, 0)),
            pl.BlockSpec((BTG, 1), lambda i: (i, 0)),
            pl.BlockSpec((2, 2, 16), lambda i: (i, 0, 0)),
            pl.BlockSpec((BTG, DP), lambda i: (i, 0)),
        ],
        out_shape=[
            jax.ShapeDtypeStruct((T, 1), jnp.int32),
            jax.ShapeDtypeStruct((T, 1), jnp.int32),
            jax.ShapeDtypeStruct((T, 1), jnp.float32),
            jax.ShapeDtypeStruct((T, 1), jnp.float32),
            jax.ShapeDtypeStruct((T // APW, 2, 16), jnp.int32),
            jax.ShapeDtypeStruct((T, DP), jnp.int32),
        ],
    )(x, gate_weight, bias2)


_SC_MESH = plsc.VectorSubcoreMesh(core_axis_name="c", subcore_axis_name="s",
                                  num_cores=2, num_subcores=16)


def _wid():
    return lax.axis_index("s") * 2 + lax.axis_index("c")


def _sc_route_body(ev_hbm, cnt_hbm, x_hbm, pos_hbm, be_hbm, xs_hbm,
                   ev_v, cnt_all_v, idx0_v, idx1_v, xrows_v, be_v, sem):
    wid = _wid()
    iota16 = lax.iota(jnp.int32, 16)
    pltpu.sync_copy(cnt_hbm, cnt_all_v)
    pltpu.sync_copy(ev_hbm.at[pl.ds(wid * APW, APW)], ev_v)

    tot = jnp.zeros((16,), jnp.int32)
    base_mine = jnp.zeros((16,), jnp.int32)
    for w in range(NW):
        row = cnt_all_v[pl.ds((w % 16) * 32 + (w // 16) * 16, 16)]
        tot = tot + row
        base_mine = base_mine + jnp.where(w < wid, row, 0)
    padded = (tot + (BT - 1)) & (-BT)
    ends = plsc.cumsum(padded)
    off = ends - padded
    base_vec = off + base_mine

    for c in range(APW // 16):
        ch = ev_v[pl.ds(c * 16, 16)]
        poschunk = jnp.zeros((16,), jnp.int32)
        for e in range(E):
            m = ch == e
            mi = jnp.where(m, 1, 0)
            pc = plsc.cumsum(mi)
            base_e = jnp.sum(jnp.where(iota16 == e, base_vec, 0))
            poschunk = jnp.where(m, base_e + pc - 1, poschunk)
            base_vec = base_vec + jnp.where(iota16 == e, jnp.sum(mi), 0)
        if c < (APW // 32):
            idx0_v[pl.ds(c * 16, 16)] = poschunk
        else:
            idx1_v[pl.ds(c * 16 - APW // 2, 16)] = poschunk
    pltpu.sync_copy(idx0_v, pos_hbm.at[pl.ds(wid * APW, APW // 2)])
    pltpu.sync_copy(idx1_v, pos_hbm.at[pl.ds(wid * APW + APW // 2, APW // 2)])

    tok0 = (wid % 16) * APW
    half = APW // 2
    for h, idx_v in ((0, idx0_v), (1, idx1_v)):
        pltpu.sync_copy(x_hbm.at[pl.ds(tok0 + h * half, half)], xrows_v)
        pltpu.async_copy(xrows_v, xs_hbm.at[idx_v], sem).wait()

    @pl.when(wid == 0)
    def _():
        bev0 = jnp.zeros((16,), jnp.int32)
        bev1 = jnp.zeros((16,), jnp.int32)
        for e in range(E):
            end_e = jnp.sum(jnp.where(iota16 == e, ends, 0))
            bev0 = bev0 + jnp.where(iota16 * BT >= end_e, 1, 0)
            bev1 = bev1 + jnp.where((iota16 + 16) * BT >= end_e, 1, 0)
        be_v[pl.ds(0, 16)] = bev0
        be_v[pl.ds(16, 16)] = bev1
        pltpu.sync_copy(be_v, be_hbm)


_sc_route = functools.partial(
    pl.kernel,
    out_type=[
        jax.ShapeDtypeStruct((NA,), jnp.int32),
        jax.ShapeDtypeStruct((32,), jnp.int32),
        jax.ShapeDtypeStruct((PN, DP), jnp.int32),
    ],
    mesh=_SC_MESH,
    compiler_params=pltpu.CompilerParams(needs_layout_passes=False),
    scratch_types=[
        pltpu.VMEM((APW,), jnp.int32),
        pltpu.VMEM((NW * 16,), jnp.int32),
        pltpu.VMEM((APW // 2,), jnp.int32),
        pltpu.VMEM((APW // 2,), jnp.int32),
        pltpu.VMEM((APW // 2, DP), jnp.int32),
        pltpu.VMEM((32,), jnp.int32),
        pltpu.SemaphoreType.DMA,
    ],
)(_sc_route_body)


def _group_mlp_body(be_ref, x_ref, wg_ref, wu_ref, wd_ref, y_ref):
    e = be_ref[pl.program_id(0)]

    @pl.when(e < E)
    def _():
        x = _unpack_rows(x_ref[...])
        g = jax.lax.dot_general(x, wg_ref[0], (((1,), (1,)), ((), ())),
                                preferred_element_type=jnp.float32)
        u = jax.lax.dot_general(x, wu_ref[0], (((1,), (1,)), ((), ())),
                                preferred_element_type=jnp.float32)
        h = _silu(g) * u
        y = jax.lax.dot_general(h, wd_ref[0], (((1,), (1,)), ((), ())),
                                preferred_element_type=jnp.float32)
        y_ref[...] = _pack_rows(y)


def _group_mlp(be, xs, Wg, Wu, Wd):
    def wmap(i, s):
        return (jnp.minimum(s[i], E - 1), 0, 0)

    grid_spec = pltpu.PrefetchScalarGridSpec(
        num_scalar_prefetch=1,
        grid=(NB,),
        in_specs=[
            pl.BlockSpec((BT, DP), lambda i, s: (i, 0)),
            pl.BlockSpec((1, FF, D), wmap),
            pl.BlockSpec((1, FF, D), wmap),
            pl.BlockSpec((1, D, FF), wmap),
        ],
        out_specs=pl.BlockSpec((BT, DP), lambda i, s: (i, 0)),
    )
    return pl.pallas_call(
        _group_mlp_body,
        grid_spec=grid_spec,
        out_shape=jax.ShapeDtypeStruct((PN, DP), jnp.int32),
    )(be, xs, Wg, Wu, Wd)


def _sc_gather_body(pos_hbm, ys_hbm, yg0_hbm, yg1_hbm, idx_v, rows_v, sem):
    wid = _wid()
    for s, out_hbm in ((0, yg0_hbm), (1, yg1_hbm)):
        pltpu.sync_copy(pos_hbm.at[pl.ds(s * T + wid * TPW, TPW)], idx_v)
        pltpu.async_copy(ys_hbm.at[idx_v], rows_v, sem).wait()
        pltpu.sync_copy(rows_v, out_hbm.at[pl.ds(wid * TPW, TPW)])


_sc_gather = functools.partial(
    pl.kernel,
    out_type=[
        jax.ShapeDtypeStruct((T, DP), jnp.int32),
        jax.ShapeDtypeStruct((T, DP), jnp.int32),
    ],
    mesh=_SC_MESH,
    compiler_params=pltpu.CompilerParams(needs_layout_passes=False),
    scratch_types=[
        pltpu.VMEM((TPW,), jnp.int32),
        pltpu.VMEM((TPW, DP), jnp.int32),
        pltpu.SemaphoreType.DMA,
    ],
)(_sc_gather_body)


def _shared_body(x_ref, wgs_ref, wus_ref, wds_ref, out_ref):
    x = x_ref[...]
    gs = jax.lax.dot_general(x, wgs_ref[...], (((1,), (1,)), ((), ())),
                             preferred_element_type=jnp.float32)
    us = jax.lax.dot_general(x, wus_ref[...], (((1,), (1,)), ((), ())),
                             preferred_element_type=jnp.float32)
    hs = _silu(gs) * us
    out_ref[...] = jax.lax.dot_general(hs, wds_ref[...], (((1,), (1,)), ((), ())),
                                       preferred_element_type=jnp.float32)


def _shared_mlp(x, Wg_s, Wu_s, Wd_s):
    BTS = 512
    return pl.pallas_call(
        _shared_body,
        grid=(T // BTS,),
        in_specs=[
            pl.BlockSpec((BTS, D), lambda i: (i, 0)),
            pl.BlockSpec(Wg_s.shape, lambda i: (0, 0)),
            pl.BlockSpec(Wu_s.shape, lambda i: (0, 0)),
            pl.BlockSpec(Wd_s.shape, lambda i: (0, 0)),
        ],
        out_specs=pl.BlockSpec((BTS, D), lambda i: (i, 0)),
        out_shape=jax.ShapeDtypeStruct((T, D), jnp.float32),
    )(x, Wg_s, Wu_s, Wd_s)


def _combine_body(sh_ref, y0_ref, y1_ref, w1_ref, w2_ref, out_ref):
    out_ref[...] = (sh_ref[...] + w1_ref[...] * _unpack_rows(y0_ref[...])
                    + w2_ref[...] * _unpack_rows(y1_ref[...]))


def _combine(shared, yg0, yg1, w1, w2):
    BTC = 512
    return pl.pallas_call(
        _combine_body,
        grid=(T // BTC,),
        in_specs=[
            pl.BlockSpec((BTC, D), lambda i: (i, 0)),
            pl.BlockSpec((BTC, DP), lambda i: (i, 0)),
            pl.BlockSpec((BTC, DP), lambda i: (i, 0)),
            pl.BlockSpec((BTC, 1), lambda i: (i, 0)),
            pl.BlockSpec((BTC, 1), lambda i: (i, 0)),
        ],
        out_specs=pl.BlockSpec((BTC, D), lambda i: (i, 0)),
        out_shape=jax.ShapeDtypeStruct((T, D), jnp.float32),
    )(shared, yg0, yg1, w1, w2)


def kernel(hidden_states, gate_weight, e_score_correction_bias, Wg, Wu, Wd,
           Wg_s, Wu_s, Wd_s):
    orig_shape = hidden_states.shape
    x = hidden_states.reshape(-1, orig_shape[-1])
    bias2 = e_score_correction_bias.reshape(1, E)

    i1, i2, w1, w2, cnt3, xb = _gate(x, gate_weight, bias2)
    ev = jnp.concatenate([i1, i2], axis=0).reshape(NA)
    cnt = cnt3.reshape(NW * 16)
    pos, be, xs = _sc_route(ev, cnt, xb)
    shared = _shared_mlp(x, Wg_s, Wu_s, Wd_s)
    ys = _group_mlp(be, xs, Wg, Wu, Wd)
    yg0, yg1 = _sc_gather(pos, ys)
    out = _combine(shared, yg0, yg1, w1, w2)
    return out.reshape(orig_shape)


# speedup vs baseline: 1.1749x; 1.0151x over previous
import functools

import jax
import jax.numpy as jnp
from jax import lax
from jax.experimental import pallas as pl
from jax.experimental.pallas import tpu as pltpu
from jax.experimental.pallas import tpu_sc as plsc

E = 8
TOP_K = 2
T = 2048
D = 1024
FF = 512
NEG = -1.0e30

BT = 256
NA = T * TOP_K
PN = NA + E * BT
NB = PN // BT
NW = 32
APW = NA // NW
TPW = T // NW
DP = D // 2
MHI = -65536
RND = 32768


def _silu(v):
    return v / (1.0 + jnp.exp(-v))


def _pack_rows(v):
    b = jax.lax.bitcast_convert_type(v, jnp.int32)
    hi = (b[:, :DP] + RND) & MHI
    lo = jax.lax.shift_right_logical(b[:, DP:] + RND, 16)
    return hi | lo


def _unpack_rows(p):
    hi = jax.lax.bitcast_convert_type(p & MHI, jnp.float32)
    lo = jax.lax.bitcast_convert_type(jax.lax.shift_left(p, 16), jnp.float32)
    return jnp.concatenate([hi, lo], axis=1)


def _subcnt(idx_col):
    eq = idx_col == jax.lax.broadcasted_iota(jnp.int32, (idx_col.shape[0], 16), 1)
    return jnp.sum(jnp.where(eq, 1, 0), axis=0, keepdims=True)


def _gate_body(x_ref, gw_ref, bias_ref, wgs_ref, wus_ref, wds_ref,
               i1_ref, i2_ref, w1_ref, w2_ref, cnt_ref, xb_ref, sh_ref):
    logits = jax.lax.dot_general(x_ref[...], gw_ref[...], (((1,), (1,)), ((), ())),
                                 preferred_element_type=jnp.float32)
    logits = logits + bias_ref[...]
    iota = jax.lax.broadcasted_iota(jnp.int32, logits.shape, 1)
    m1 = jnp.max(logits, axis=1, keepdims=True)
    i1 = jnp.min(jnp.where(logits == m1, iota, E), axis=1, keepdims=True)
    masked = jnp.where(iota == i1, NEG, logits)
    m2 = jnp.max(masked, axis=1, keepdims=True)
    i2 = jnp.min(jnp.where(masked == m2, iota, E), axis=1, keepdims=True)
    e2 = jnp.exp(m2 - m1)
    w1 = 1.0 / (1.0 + e2)
    i1_ref[...] = i1
    i2_ref[...] = i2
    w1_ref[...] = w1
    w2_ref[...] = 1.0 - w1
    h = jnp.concatenate([
        _subcnt(i1[0:APW, :]), _subcnt(i2[0:APW, :]),
        _subcnt(i1[APW:2 * APW, :]), _subcnt(i2[APW:2 * APW, :]),
    ], axis=0)
    cnt_ref[...] = h.reshape(2, 2, 16)
    xb_ref[...] = _pack_rows(x_ref[...])
    x = x_ref[...]
    gs = jax.lax.dot_general(x, wgs_ref[...], (((1,), (1,)), ((), ())),
                             preferred_element_type=jnp.float32)
    us = jax.lax.dot_general(x, wus_ref[...], (((1,), (1,)), ((), ())),
                             preferred_element_type=jnp.float32)
    hs = _silu(gs) * us
    sh = jax.lax.dot_general(hs, wds_ref[...], (((1,), (1,)), ((), ())),
                             preferred_element_type=jnp.float32)
    sh_ref[...] = _pack_rows(sh)


def _gate(x, gate_weight, bias2, Wg_s, Wu_s, Wd_s):
    BTG = 256
    return pl.pallas_call(
        _gate_body,
        grid=(T // BTG,),
        in_specs=[
            pl.BlockSpec((BTG, D), lambda i: (i, 0)),
            pl.BlockSpec((E, D), lambda i: (0, 0)),
            pl.BlockSpec((1, E), lambda i: (0, 0)),
            pl.BlockSpec(Wg_s.shape, lambda i: (0, 0)),
            pl.BlockSpec(Wu_s.shape, lambda i: (0, 0)),
            pl.BlockSpec(Wd_s.shape, lambda i: (0, 0)),
        ],
        out_specs=[
            pl.BlockSpec((BTG, 1), lambda i: (i, 0)),
            pl.BlockSpec((BTG, 1), lambda i: (i, 0)),
            pl.BlockSpec((BTG, 1), lambda i: (i, 0)),
            pl.BlockSpec((BTG, 1), lambda i: (i, 0)),
            pl.BlockSpec((2, 2, 16), lambda i: (i, 0, 0)),
            pl.BlockSpec((BTG, DP), lambda i: (i, 0)),
            pl.BlockSpec((BTG, DP), lambda i: (i, 0)),
        ],
        out_shape=[
            jax.ShapeDtypeStruct((T, 1), jnp.int32),
            jax.ShapeDtypeStruct((T, 1), jnp.int32),
            jax.ShapeDtypeStruct((T, 1), jnp.float32),
            jax.ShapeDtypeStruct((T, 1), jnp.float32),
            jax.ShapeDtypeStruct((T // APW, 2, 16), jnp.int32),
            jax.ShapeDtypeStruct((T, DP), jnp.int32),
            jax.ShapeDtypeStruct((T, DP), jnp.int32),
        ],
    )(x, gate_weight, bias2, Wg_s, Wu_s, Wd_s)


_SC_MESH = plsc.VectorSubcoreMesh(core_axis_name="c", subcore_axis_name="s",
                                  num_cores=2, num_subcores=16)


def _wid():
    return lax.axis_index("s") * 2 + lax.axis_index("c")


def _sc_route_body(ev_hbm, cnt_hbm, x_hbm, pos_hbm, be_hbm, xs_hbm,
                   ev_v, cnt_all_v, idx0_v, idx1_v, xrows_v, be_v, sem):
    wid = _wid()
    iota16 = lax.iota(jnp.int32, 16)
    pltpu.sync_copy(cnt_hbm, cnt_all_v)
    pltpu.sync_copy(ev_hbm.at[pl.ds(wid * APW, APW)], ev_v)

    tot = jnp.zeros((16,), jnp.int32)
    base_mine = jnp.zeros((16,), jnp.int32)
    for w in range(NW):
        row = cnt_all_v[pl.ds((w % 16) * 32 + (w // 16) * 16, 16)]
        tot = tot + row
        base_mine = base_mine + jnp.where(w < wid, row, 0)
    padded = (tot + (BT - 1)) & (-BT)
    ends = plsc.cumsum(padded)
    off = ends - padded
    base_vec = off + base_mine

    for c in range(APW // 16):
        ch = ev_v[pl.ds(c * 16, 16)]
        poschunk = jnp.zeros((16,), jnp.int32)
        for e in range(E):
            m = ch == e
            mi = jnp.where(m, 1, 0)
            pc = plsc.cumsum(mi)
            base_e = jnp.sum(jnp.where(iota16 == e, base_vec, 0))
            poschunk = jnp.where(m, base_e + pc - 1, poschunk)
            base_vec = base_vec + jnp.where(iota16 == e, jnp.sum(mi), 0)
        if c < (APW // 32):
            idx0_v[pl.ds(c * 16, 16)] = poschunk
        else:
            idx1_v[pl.ds(c * 16 - APW // 2, 16)] = poschunk
    pltpu.sync_copy(idx0_v, pos_hbm.at[pl.ds(wid * APW, APW // 2)])
    pltpu.sync_copy(idx1_v, pos_hbm.at[pl.ds(wid * APW + APW // 2, APW // 2)])

    tok0 = (wid % 16) * APW
    half = APW // 2
    for h, idx_v in ((0, idx0_v), (1, idx1_v)):
        pltpu.sync_copy(x_hbm.at[pl.ds(tok0 + h * half, half)], xrows_v)
        pltpu.async_copy(xrows_v, xs_hbm.at[idx_v], sem).wait()

    @pl.when(wid == 0)
    def _():
        bev0 = jnp.zeros((16,), jnp.int32)
        bev1 = jnp.zeros((16,), jnp.int32)
        for e in range(E):
            end_e = jnp.sum(jnp.where(iota16 == e, ends, 0))
            bev0 = bev0 + jnp.where(iota16 * BT >= end_e, 1, 0)
            bev1 = bev1 + jnp.where((iota16 + 16) * BT >= end_e, 1, 0)
        be_v[pl.ds(0, 16)] = bev0
        be_v[pl.ds(16, 16)] = bev1
        pltpu.sync_copy(be_v, be_hbm)


_sc_route = functools.partial(
    pl.kernel,
    out_type=[
        jax.ShapeDtypeStruct((NA,), jnp.int32),
        jax.ShapeDtypeStruct((32,), jnp.int32),
        jax.ShapeDtypeStruct((PN, DP), jnp.int32),
    ],
    mesh=_SC_MESH,
    compiler_params=pltpu.CompilerParams(needs_layout_passes=False),
    scratch_types=[
        pltpu.VMEM((APW,), jnp.int32),
        pltpu.VMEM((NW * 16,), jnp.int32),
        pltpu.VMEM((APW // 2,), jnp.int32),
        pltpu.VMEM((APW // 2,), jnp.int32),
        pltpu.VMEM((APW // 2, DP), jnp.int32),
        pltpu.VMEM((32,), jnp.int32),
        pltpu.SemaphoreType.DMA,
    ],
)(_sc_route_body)


def _group_mlp_body(be_ref, x_ref, wg_ref, wu_ref, wd_ref, y_ref):
    e = be_ref[pl.program_id(0)]

    @pl.when(e < E)
    def _():
        x = _unpack_rows(x_ref[...])
        g = jax.lax.dot_general(x, wg_ref[0], (((1,), (1,)), ((), ())),
                                preferred_element_type=jnp.float32)
        u = jax.lax.dot_general(x, wu_ref[0], (((1,), (1,)), ((), ())),
                                preferred_element_type=jnp.float32)
        h = _silu(g) * u
        y = jax.lax.dot_general(h, wd_ref[0], (((1,), (1,)), ((), ())),
                                preferred_element_type=jnp.float32)
        y_ref[...] = _pack_rows(y)


def _group_mlp(be, xs, Wg, Wu, Wd):
    def wmap(i, s):
        return (jnp.minimum(s[i], E - 1), 0, 0)

    grid_spec = pltpu.PrefetchScalarGridSpec(
        num_scalar_prefetch=1,
        grid=(NB,),
        in_specs=[
            pl.BlockSpec((BT, DP), lambda i, s: (i, 0)),
            pl.BlockSpec((1, FF, D), wmap),
            pl.BlockSpec((1, FF, D), wmap),
            pl.BlockSpec((1, D, FF), wmap),
        ],
        out_specs=pl.BlockSpec((BT, DP), lambda i, s: (i, 0)),
    )
    return pl.pallas_call(
        _group_mlp_body,
        grid_spec=grid_spec,
        out_shape=jax.ShapeDtypeStruct((PN, DP), jnp.int32),
    )(be, xs, Wg, Wu, Wd)


def _sc_gather_body(pos_hbm, ys_hbm, yg0_hbm, yg1_hbm, idx_v, rows_v, sem):
    wid = _wid()
    for s, out_hbm in ((0, yg0_hbm), (1, yg1_hbm)):
        pltpu.sync_copy(pos_hbm.at[pl.ds(s * T + wid * TPW, TPW)], idx_v)
        pltpu.async_copy(ys_hbm.at[idx_v], rows_v, sem).wait()
        pltpu.sync_copy(rows_v, out_hbm.at[pl.ds(wid * TPW, TPW)])


_sc_gather = functools.partial(
    pl.kernel,
    out_type=[
        jax.ShapeDtypeStruct((T, DP), jnp.int32),
        jax.ShapeDtypeStruct((T, DP), jnp.int32),
    ],
    mesh=_SC_MESH,
    compiler_params=pltpu.CompilerParams(needs_layout_passes=False),
    scratch_types=[
        pltpu.VMEM((TPW,), jnp.int32),
        pltpu.VMEM((TPW, DP), jnp.int32),
        pltpu.SemaphoreType.DMA,
    ],
)(_sc_gather_body)


def _combine_body(sh_ref, y0_ref, y1_ref, w1_ref, w2_ref, out_ref):
    out_ref[...] = (_unpack_rows(sh_ref[...])
                    + w1_ref[...] * _unpack_rows(y0_ref[...])
                    + w2_ref[...] * _unpack_rows(y1_ref[...]))


def _combine(shared, yg0, yg1, w1, w2):
    BTC = 512
    return pl.pallas_call(
        _combine_body,
        grid=(T // BTC,),
        in_specs=[
            pl.BlockSpec((BTC, DP), lambda i: (i, 0)),
            pl.BlockSpec((BTC, DP), lambda i: (i, 0)),
            pl.BlockSpec((BTC, DP), lambda i: (i, 0)),
            pl.BlockSpec((BTC, 1), lambda i: (i, 0)),
            pl.BlockSpec((BTC, 1), lambda i: (i, 0)),
        ],
        out_specs=pl.BlockSpec((BTC, D), lambda i: (i, 0)),
        out_shape=jax.ShapeDtypeStruct((T, D), jnp.float32),
    )(shared, yg0, yg1, w1, w2)


def kernel(hidden_states, gate_weight, e_score_correction_bias, Wg, Wu, Wd,
           Wg_s, Wu_s, Wd_s):
    orig_shape = hidden_states.shape
    x = hidden_states.reshape(-1, orig_shape[-1])
    bias2 = e_score_correction_bias.reshape(1, E)

    i1, i2, w1, w2, cnt3, xb, shp = _gate(x, gate_weight, bias2, Wg_s, Wu_s, Wd_s)
    ev = jnp.concatenate([i1, i2], axis=0).reshape(NA)
    cnt = cnt3.reshape(NW * 16)
    pos, be, xs = _sc_route(ev, cnt, xb)
    ys = _group_mlp(be, xs, Wg, Wu, Wd)
    yg0, yg1 = _sc_gather(pos, ys)
    out = _combine(shp, yg0, yg1, w1, w2)
    return out.reshape(orig_shape)


# speedup vs baseline: 1.1758x; 1.0007x over previous
import functools

import jax
import jax.numpy as jnp
from jax import lax
from jax.experimental import pallas as pl
from jax.experimental.pallas import tpu as pltpu
from jax.experimental.pallas import tpu_sc as plsc

E = 8
TOP_K = 2
T = 2048
D = 1024
FF = 512
NEG = -1.0e30

BT = 256
NA = T * TOP_K
PN = NA + E * BT
NB = PN // BT
NW = 32
APW = NA // NW
TPW = T // NW
DP = D // 2
MHI = -65536
RND = 32768


def _silu(v):
    return v / (1.0 + jnp.exp(-v))


def _pack_rows(v):
    b = jax.lax.bitcast_convert_type(v, jnp.int32)
    hi = (b[:, :DP] + RND) & MHI
    lo = jax.lax.shift_right_logical(b[:, DP:] + RND, 16)
    return hi | lo


def _unpack_rows(p):
    hi = jax.lax.bitcast_convert_type(p & MHI, jnp.float32)
    lo = jax.lax.bitcast_convert_type(jax.lax.shift_left(p, 16), jnp.float32)
    return jnp.concatenate([hi, lo], axis=1)


def _subcnt(idx_col):
    eq = idx_col == jax.lax.broadcasted_iota(jnp.int32, (idx_col.shape[0], 16), 1)
    return jnp.sum(jnp.where(eq, 1, 0), axis=0, keepdims=True)


def _gate_body(x_ref, gw_ref, bias_ref, wgs_ref, wus_ref, wds_ref,
               i1_ref, i2_ref, w1_ref, w2_ref, cnt_ref, xb_ref, sh_ref):
    logits = jax.lax.dot_general(x_ref[...], gw_ref[...], (((1,), (1,)), ((), ())),
                                 preferred_element_type=jnp.float32)
    logits = logits + bias_ref[...]
    iota = jax.lax.broadcasted_iota(jnp.int32, logits.shape, 1)
    m1 = jnp.max(logits, axis=1, keepdims=True)
    i1 = jnp.min(jnp.where(logits == m1, iota, E), axis=1, keepdims=True)
    masked = jnp.where(iota == i1, NEG, logits)
    m2 = jnp.max(masked, axis=1, keepdims=True)
    i2 = jnp.min(jnp.where(masked == m2, iota, E), axis=1, keepdims=True)
    e2 = jnp.exp(m2 - m1)
    w1 = 1.0 / (1.0 + e2)
    i1_ref[...] = i1
    i2_ref[...] = i2
    w1_ref[...] = w1
    w2_ref[...] = 1.0 - w1
    h = jnp.concatenate([
        _subcnt(i1[0:APW, :]), _subcnt(i2[0:APW, :]),
        _subcnt(i1[APW:2 * APW, :]), _subcnt(i2[APW:2 * APW, :]),
    ], axis=0)
    cnt_ref[...] = h.reshape(2, 2, 16)
    xb_ref[...] = _pack_rows(x_ref[...])
    xb16 = x_ref[...].astype(jnp.bfloat16)
    gs = jax.lax.dot_general(xb16, wgs_ref[...].astype(jnp.bfloat16),
                             (((1,), (1,)), ((), ())),
                             preferred_element_type=jnp.float32)
    us = jax.lax.dot_general(xb16, wus_ref[...].astype(jnp.bfloat16),
                             (((1,), (1,)), ((), ())),
                             preferred_element_type=jnp.float32)
    hs = (_silu(gs) * us).astype(jnp.bfloat16)
    sh = jax.lax.dot_general(hs, wds_ref[...].astype(jnp.bfloat16),
                             (((1,), (1,)), ((), ())),
                             preferred_element_type=jnp.float32)
    sh_ref[...] = _pack_rows(sh)


def _gate(x, gate_weight, bias2, Wg_s, Wu_s, Wd_s):
    BTG = 256
    return pl.pallas_call(
        _gate_body,
        grid=(T // BTG,),
        in_specs=[
            pl.BlockSpec((BTG, D), lambda i: (i, 0)),
            pl.BlockSpec((E, D), lambda i: (0, 0)),
            pl.BlockSpec((1, E), lambda i: (0, 0)),
            pl.BlockSpec(Wg_s.shape, lambda i: (0, 0)),
            pl.BlockSpec(Wu_s.shape, lambda i: (0, 0)),
            pl.BlockSpec(Wd_s.shape, lambda i: (0, 0)),
        ],
        out_specs=[
            pl.BlockSpec((BTG, 1), lambda i: (i, 0)),
            pl.BlockSpec((BTG, 1), lambda i: (i, 0)),
            pl.BlockSpec((BTG, 1), lambda i: (i, 0)),
            pl.BlockSpec((BTG, 1), lambda i: (i, 0)),
            pl.BlockSpec((2, 2, 16), lambda i: (i, 0, 0)),
            pl.BlockSpec((BTG, DP), lambda i: (i, 0)),
            pl.BlockSpec((BTG, DP), lambda i: (i, 0)),
        ],
        out_shape=[
            jax.ShapeDtypeStruct((T, 1), jnp.int32),
            jax.ShapeDtypeStruct((T, 1), jnp.int32),
            jax.ShapeDtypeStruct((T, 1), jnp.float32),
            jax.ShapeDtypeStruct((T, 1), jnp.float32),
            jax.ShapeDtypeStruct((T // APW, 2, 16), jnp.int32),
            jax.ShapeDtypeStruct((T, DP), jnp.int32),
            jax.ShapeDtypeStruct((T, DP), jnp.int32),
        ],
    )(x, gate_weight, bias2, Wg_s, Wu_s, Wd_s)


_SC_MESH = plsc.VectorSubcoreMesh(core_axis_name="c", subcore_axis_name="s",
                                  num_cores=2, num_subcores=16)


def _wid():
    return lax.axis_index("s") * 2 + lax.axis_index("c")


def _sc_route_body(ev_hbm, cnt_hbm, x_hbm, pos_hbm, be_hbm, xs_hbm,
                   ev_v, cnt_all_v, idx0_v, idx1_v, xrows_v, be_v, sem):
    wid = _wid()
    iota16 = lax.iota(jnp.int32, 16)
    pltpu.sync_copy(cnt_hbm, cnt_all_v)
    pltpu.sync_copy(ev_hbm.at[pl.ds(wid * APW, APW)], ev_v)

    tot = jnp.zeros((16,), jnp.int32)
    base_mine = jnp.zeros((16,), jnp.int32)
    for w in range(NW):
        row = cnt_all_v[pl.ds((w % 16) * 32 + (w // 16) * 16, 16)]
        tot = tot + row
        base_mine = base_mine + jnp.where(w < wid, row, 0)
    padded = (tot + (BT - 1)) & (-BT)
    ends = plsc.cumsum(padded)
    off = ends - padded
    base_vec = off + base_mine

    for c in range(APW // 16):
        ch = ev_v[pl.ds(c * 16, 16)]
        poschunk = jnp.zeros((16,), jnp.int32)
        for e in range(E):
            m = ch == e
            mi = jnp.where(m, 1, 0)
            pc = plsc.cumsum(mi)
            base_e = jnp.sum(jnp.where(iota16 == e, base_vec, 0))
            poschunk = jnp.where(m, base_e + pc - 1, poschunk)
            base_vec = base_vec + jnp.where(iota16 == e, jnp.sum(mi), 0)
        if c < (APW // 32):
            idx0_v[pl.ds(c * 16, 16)] = poschunk
        else:
            idx1_v[pl.ds(c * 16 - APW // 2, 16)] = poschunk
    pltpu.sync_copy(idx0_v, pos_hbm.at[pl.ds(wid * APW, APW // 2)])
    pltpu.sync_copy(idx1_v, pos_hbm.at[pl.ds(wid * APW + APW // 2, APW // 2)])

    tok0 = (wid % 16) * APW
    half = APW // 2
    for h, idx_v in ((0, idx0_v), (1, idx1_v)):
        pltpu.sync_copy(x_hbm.at[pl.ds(tok0 + h * half, half)], xrows_v)
        pltpu.async_copy(xrows_v, xs_hbm.at[idx_v], sem).wait()

    @pl.when(wid == 0)
    def _():
        bev0 = jnp.zeros((16,), jnp.int32)
        bev1 = jnp.zeros((16,), jnp.int32)
        for e in range(E):
            end_e = jnp.sum(jnp.where(iota16 == e, ends, 0))
            bev0 = bev0 + jnp.where(iota16 * BT >= end_e, 1, 0)
            bev1 = bev1 + jnp.where((iota16 + 16) * BT >= end_e, 1, 0)
        be_v[pl.ds(0, 16)] = bev0
        be_v[pl.ds(16, 16)] = bev1
        pltpu.sync_copy(be_v, be_hbm)


_sc_route = functools.partial(
    pl.kernel,
    out_type=[
        jax.ShapeDtypeStruct((NA,), jnp.int32),
        jax.ShapeDtypeStruct((32,), jnp.int32),
        jax.ShapeDtypeStruct((PN, DP), jnp.int32),
    ],
    mesh=_SC_MESH,
    compiler_params=pltpu.CompilerParams(needs_layout_passes=False),
    scratch_types=[
        pltpu.VMEM((APW,), jnp.int32),
        pltpu.VMEM((NW * 16,), jnp.int32),
        pltpu.VMEM((APW // 2,), jnp.int32),
        pltpu.VMEM((APW // 2,), jnp.int32),
        pltpu.VMEM((APW // 2, DP), jnp.int32),
        pltpu.VMEM((32,), jnp.int32),
        pltpu.SemaphoreType.DMA,
    ],
)(_sc_route_body)


def _group_mlp_body(be_ref, x_ref, wg_ref, wu_ref, wd_ref, y_ref):
    e = be_ref[pl.program_id(0)]

    @pl.when(e < E)
    def _():
        x = _unpack_rows(x_ref[...]).astype(jnp.bfloat16)
        g = jax.lax.dot_general(x, wg_ref[0].astype(jnp.bfloat16),
                                (((1,), (1,)), ((), ())),
                                preferred_element_type=jnp.float32)
        u = jax.lax.dot_general(x, wu_ref[0].astype(jnp.bfloat16),
                                (((1,), (1,)), ((), ())),
                                preferred_element_type=jnp.float32)
        h = (_silu(g) * u).astype(jnp.bfloat16)
        y = jax.lax.dot_general(h, wd_ref[0].astype(jnp.bfloat16),
                                (((1,), (1,)), ((), ())),
                                preferred_element_type=jnp.float32)
        y_ref[...] = _pack_rows(y)


def _group_mlp(be, xs, Wg, Wu, Wd):
    def wmap(i, s):
        return (jnp.minimum(s[i], E - 1), 0, 0)

    grid_spec = pltpu.PrefetchScalarGridSpec(
        num_scalar_prefetch=1,
        grid=(NB,),
        in_specs=[
            pl.BlockSpec((BT, DP), lambda i, s: (i, 0)),
            pl.BlockSpec((1, FF, D), wmap),
            pl.BlockSpec((1, FF, D), wmap),
            pl.BlockSpec((1, D, FF), wmap),
        ],
        out_specs=pl.BlockSpec((BT, DP), lambda i, s: (i, 0)),
    )
    return pl.pallas_call(
        _group_mlp_body,
        grid_spec=grid_spec,
        out_shape=jax.ShapeDtypeStruct((PN, DP), jnp.int32),
    )(be, xs, Wg, Wu, Wd)


def _sc_gather_body(pos_hbm, ys_hbm, yg0_hbm, yg1_hbm, idx_v, rows_v, sem):
    wid = _wid()
    for s, out_hbm in ((0, yg0_hbm), (1, yg1_hbm)):
        pltpu.sync_copy(pos_hbm.at[pl.ds(s * T + wid * TPW, TPW)], idx_v)
        pltpu.async_copy(ys_hbm.at[idx_v], rows_v, sem).wait()
        pltpu.sync_copy(rows_v, out_hbm.at[pl.ds(wid * TPW, TPW)])


_sc_gather = functools.partial(
    pl.kernel,
    out_type=[
        jax.ShapeDtypeStruct((T, DP), jnp.int32),
        jax.ShapeDtypeStruct((T, DP), jnp.int32),
    ],
    mesh=_SC_MESH,
    compiler_params=pltpu.CompilerParams(needs_layout_passes=False),
    scratch_types=[
        pltpu.VMEM((TPW,), jnp.int32),
        pltpu.VMEM((TPW, DP), jnp.int32),
        pltpu.SemaphoreType.DMA,
    ],
)(_sc_gather_body)


def _combine_body(sh_ref, y0_ref, y1_ref, w1_ref, w2_ref, out_ref):
    out_ref[...] = (_unpack_rows(sh_ref[...])
                    + w1_ref[...] * _unpack_rows(y0_ref[...])
                    + w2_ref[...] * _unpack_rows(y1_ref[...]))


def _combine(shared, yg0, yg1, w1, w2):
    BTC = 512
    return pl.pallas_call(
        _combine_body,
        grid=(T // BTC,),
        in_specs=[
            pl.BlockSpec((BTC, DP), lambda i: (i, 0)),
            pl.BlockSpec((BTC, DP), lambda i: (i, 0)),
            pl.BlockSpec((BTC, DP), lambda i: (i, 0)),
            pl.BlockSpec((BTC, 1), lambda i: (i, 0)),
            pl.BlockSpec((BTC, 1), lambda i: (i, 0)),
        ],
        out_specs=pl.BlockSpec((BTC, D), lambda i: (i, 0)),
        out_shape=jax.ShapeDtypeStruct((T, D), jnp.float32),
    )(shared, yg0, yg1, w1, w2)


def kernel(hidden_states, gate_weight, e_score_correction_bias, Wg, Wu, Wd,
           Wg_s, Wu_s, Wd_s):
    orig_shape = hidden_states.shape
    x = hidden_states.reshape(-1, orig_shape[-1])
    bias2 = e_score_correction_bias.reshape(1, E)

    i1, i2, w1, w2, cnt3, xb, shp = _gate(x, gate_weight, bias2, Wg_s, Wu_s, Wd_s)
    ev = jnp.concatenate([i1, i2], axis=0).reshape(NA)
    cnt = cnt3.reshape(NW * 16)
    pos, be, xs = _sc_route(ev, cnt, xb)
    ys = _group_mlp(be, xs, Wg, Wu, Wd)
    yg0, yg1 = _sc_gather(pos, ys)
    out = _combine(shp, yg0, yg1, w1, w2)
    return out.reshape(orig_shape)


# speedup vs baseline: 1.1800x; 1.0036x over previous
import functools

import jax
import jax.numpy as jnp
from jax import lax
from jax.experimental import pallas as pl
from jax.experimental.pallas import tpu as pltpu
from jax.experimental.pallas import tpu_sc as plsc

E = 8
TOP_K = 2
T = 2048
D = 1024
FF = 512
NEG = -1.0e30

BT = 256
NA = T * TOP_K
PN = NA + E * BT
NB = PN // BT
NW = 32
APW = NA // NW
TPW = T // NW
DP = D // 2
MHI = -65536
RND = 32768


def _silu(v):
    return v / (1.0 + jnp.exp(-v))


def _pack_rows(v):
    b = jax.lax.bitcast_convert_type(v, jnp.int32)
    hi = (b[:, :DP] + RND) & MHI
    lo = jax.lax.shift_right_logical(b[:, DP:] + RND, 16)
    return hi | lo


def _unpack_rows(p):
    hi = jax.lax.bitcast_convert_type(p & MHI, jnp.float32)
    lo = jax.lax.bitcast_convert_type(jax.lax.shift_left(p, 16), jnp.float32)
    return jnp.concatenate([hi, lo], axis=1)


def _subcnt(idx_col):
    eq = idx_col == jax.lax.broadcasted_iota(jnp.int32, (idx_col.shape[0], 16), 1)
    return jnp.sum(jnp.where(eq, 1, 0), axis=0, keepdims=True)


def _gate_body(x_ref, gw_ref, bias_ref, wgs_ref, wus_ref, wds_ref,
               i1_ref, i2_ref, w1_ref, w2_ref, cnt_ref, xb_ref, sh_ref):
    logits = jax.lax.dot_general(x_ref[...], gw_ref[...], (((1,), (1,)), ((), ())),
                                 preferred_element_type=jnp.float32)
    logits = logits + bias_ref[...]
    iota = jax.lax.broadcasted_iota(jnp.int32, logits.shape, 1)
    m1 = jnp.max(logits, axis=1, keepdims=True)
    i1 = jnp.min(jnp.where(logits == m1, iota, E), axis=1, keepdims=True)
    masked = jnp.where(iota == i1, NEG, logits)
    m2 = jnp.max(masked, axis=1, keepdims=True)
    i2 = jnp.min(jnp.where(masked == m2, iota, E), axis=1, keepdims=True)
    e2 = jnp.exp(m2 - m1)
    w1 = 1.0 / (1.0 + e2)
    i1_ref[...] = i1
    i2_ref[...] = i2
    w1_ref[...] = w1
    w2_ref[...] = 1.0 - w1
    h = jnp.concatenate([
        _subcnt(i1[0:APW, :]), _subcnt(i2[0:APW, :]),
        _subcnt(i1[APW:2 * APW, :]), _subcnt(i2[APW:2 * APW, :]),
    ], axis=0)
    cnt_ref[...] = h.reshape(2, 2, 16)
    xb_ref[...] = _pack_rows(x_ref[...])
    x = x_ref[...]
    gs = jax.lax.dot_general(x, wgs_ref[...], (((1,), (1,)), ((), ())),
                             preferred_element_type=jnp.float32)
    us = jax.lax.dot_general(x, wus_ref[...], (((1,), (1,)), ((), ())),
                             preferred_element_type=jnp.float32)
    hs = _silu(gs) * us
    sh = jax.lax.dot_general(hs, wds_ref[...], (((1,), (1,)), ((), ())),
                             preferred_element_type=jnp.float32)
    sh_ref[...] = _pack_rows(sh)


def _gate(x, gate_weight, bias2, Wg_s, Wu_s, Wd_s):
    BTG = 256
    return pl.pallas_call(
        _gate_body,
        grid=(T // BTG,),
        in_specs=[
            pl.BlockSpec((BTG, D), lambda i: (i, 0)),
            pl.BlockSpec((E, D), lambda i: (0, 0)),
            pl.BlockSpec((1, E), lambda i: (0, 0)),
            pl.BlockSpec(Wg_s.shape, lambda i: (0, 0)),
            pl.BlockSpec(Wu_s.shape, lambda i: (0, 0)),
            pl.BlockSpec(Wd_s.shape, lambda i: (0, 0)),
        ],
        out_specs=[
            pl.BlockSpec((BTG, 1), lambda i: (i, 0)),
            pl.BlockSpec((BTG, 1), lambda i: (i, 0)),
            pl.BlockSpec((BTG, 1), lambda i: (i, 0)),
            pl.BlockSpec((BTG, 1), lambda i: (i, 0)),
            pl.BlockSpec((2, 2, 16), lambda i: (i, 0, 0)),
            pl.BlockSpec((BTG, DP), lambda i: (i, 0)),
            pl.BlockSpec((BTG, DP), lambda i: (i, 0)),
        ],
        out_shape=[
            jax.ShapeDtypeStruct((T, 1), jnp.int32),
            jax.ShapeDtypeStruct((T, 1), jnp.int32),
            jax.ShapeDtypeStruct((T, 1), jnp.float32),
            jax.ShapeDtypeStruct((T, 1), jnp.float32),
            jax.ShapeDtypeStruct((T // APW, 2, 16), jnp.int32),
            jax.ShapeDtypeStruct((T, DP), jnp.int32),
            jax.ShapeDtypeStruct((T, DP), jnp.int32),
        ],
    )(x, gate_weight, bias2, Wg_s, Wu_s, Wd_s)


_SC_MESH = plsc.VectorSubcoreMesh(core_axis_name="c", subcore_axis_name="s",
                                  num_cores=2, num_subcores=16)


def _wid():
    return lax.axis_index("s") * 2 + lax.axis_index("c")


def _sc_route_body(ev_hbm, cnt_hbm, x_hbm, pos_hbm, be_hbm, xs_hbm,
                   ev_v, cnt_all_v, idx0_v, idx1_v, xrows_v, be_v, sem):
    wid = _wid()
    iota16 = lax.iota(jnp.int32, 16)
    pltpu.sync_copy(cnt_hbm, cnt_all_v)
    pltpu.sync_copy(ev_hbm.at[pl.ds(wid * APW, APW)], ev_v)

    tot = jnp.zeros((16,), jnp.int32)
    base_mine = jnp.zeros((16,), jnp.int32)
    for w in range(NW):
        row = cnt_all_v[pl.ds((w % 16) * 32 + (w // 16) * 16, 16)]
        tot = tot + row
        base_mine = base_mine + jnp.where(w < wid, row, 0)
    padded = (tot + (BT - 1)) & (-BT)
    ends = plsc.cumsum(padded)
    off = ends - padded
    base_vec = off + base_mine

    for c in range(APW // 16):
        ch = ev_v[pl.ds(c * 16, 16)]
        poschunk = jnp.zeros((16,), jnp.int32)
        for e in range(E):
            m = ch == e
            mi = jnp.where(m, 1, 0)
            pc = plsc.cumsum(mi)
            base_e = jnp.sum(jnp.where(iota16 == e, base_vec, 0))
            poschunk = jnp.where(m, base_e + pc - 1, poschunk)
            base_vec = base_vec + jnp.where(iota16 == e, jnp.sum(mi), 0)
        if c < (APW // 32):
            idx0_v[pl.ds(c * 16, 16)] = poschunk
        else:
            idx1_v[pl.ds(c * 16 - APW // 2, 16)] = poschunk
    pltpu.sync_copy(idx0_v, pos_hbm.at[pl.ds(wid * APW, APW // 2)])
    pltpu.sync_copy(idx1_v, pos_hbm.at[pl.ds(wid * APW + APW // 2, APW // 2)])

    tok0 = (wid % 16) * APW
    half = APW // 2
    for h, idx_v in ((0, idx0_v), (1, idx1_v)):
        pltpu.sync_copy(x_hbm.at[pl.ds(tok0 + h * half, half)], xrows_v)
        pltpu.async_copy(xrows_v, xs_hbm.at[idx_v], sem).wait()

    @pl.when(wid == 0)
    def _():
        bev0 = jnp.zeros((16,), jnp.int32)
        bev1 = jnp.zeros((16,), jnp.int32)
        for e in range(E):
            end_e = jnp.sum(jnp.where(iota16 == e, ends, 0))
            bev0 = bev0 + jnp.where(iota16 * BT >= end_e, 1, 0)
            bev1 = bev1 + jnp.where((iota16 + 16) * BT >= end_e, 1, 0)
        be_v[pl.ds(0, 16)] = bev0
        be_v[pl.ds(16, 16)] = bev1
        pltpu.sync_copy(be_v, be_hbm)


_sc_route = functools.partial(
    pl.kernel,
    out_type=[
        jax.ShapeDtypeStruct((NA,), jnp.int32),
        jax.ShapeDtypeStruct((32,), jnp.int32),
        jax.ShapeDtypeStruct((PN, DP), jnp.int32),
    ],
    mesh=_SC_MESH,
    compiler_params=pltpu.CompilerParams(needs_layout_passes=False),
    scratch_types=[
        pltpu.VMEM((APW,), jnp.int32),
        pltpu.VMEM((NW * 16,), jnp.int32),
        pltpu.VMEM((APW // 2,), jnp.int32),
        pltpu.VMEM((APW // 2,), jnp.int32),
        pltpu.VMEM((APW // 2, DP), jnp.int32),
        pltpu.VMEM((32,), jnp.int32),
        pltpu.SemaphoreType.DMA,
    ],
)(_sc_route_body)


def _group_mlp_body(be_ref, x_ref, wg_ref, wu_ref, wd_ref, y_ref):
    e = be_ref[pl.program_id(0)]

    @pl.when(e < E)
    def _():
        x = _unpack_rows(x_ref[...])
        g = jax.lax.dot_general(x, wg_ref[0], (((1,), (1,)), ((), ())),
                                preferred_element_type=jnp.float32)
        u = jax.lax.dot_general(x, wu_ref[0], (((1,), (1,)), ((), ())),
                                preferred_element_type=jnp.float32)
        h = _silu(g) * u
        y = jax.lax.dot_general(h, wd_ref[0], (((1,), (1,)), ((), ())),
                                preferred_element_type=jnp.float32)
        y_ref[...] = _pack_rows(y)


def _group_mlp(be, xs, Wg, Wu, Wd):
    def wmap(i, s):
        return (jnp.minimum(s[i], E - 1), 0, 0)

    grid_spec = pltpu.PrefetchScalarGridSpec(
        num_scalar_prefetch=1,
        grid=(NB,),
        in_specs=[
            pl.BlockSpec((BT, DP), lambda i, s: (i, 0)),
            pl.BlockSpec((1, FF, D), wmap),
            pl.BlockSpec((1, FF, D), wmap),
            pl.BlockSpec((1, D, FF), wmap),
        ],
        out_specs=pl.BlockSpec((BT, DP), lambda i, s: (i, 0)),
    )
    return pl.pallas_call(
        _group_mlp_body,
        grid_spec=grid_spec,
        out_shape=jax.ShapeDtypeStruct((PN, DP), jnp.int32),
    )(be, xs, Wg, Wu, Wd)


def _sc_gather_body(pos_hbm, ys_hbm, yg0_hbm, yg1_hbm, idx_v, rows_v, sem):
    wid = _wid()
    for s, out_hbm in ((0, yg0_hbm), (1, yg1_hbm)):
        pltpu.sync_copy(pos_hbm.at[pl.ds(s * T + wid * TPW, TPW)], idx_v)
        pltpu.async_copy(ys_hbm.at[idx_v], rows_v, sem).wait()
        pltpu.sync_copy(rows_v, out_hbm.at[pl.ds(wid * TPW, TPW)])


_sc_gather = functools.partial(
    pl.kernel,
    out_type=[
        jax.ShapeDtypeStruct((T, DP), jnp.int32),
        jax.ShapeDtypeStruct((T, DP), jnp.int32),
    ],
    mesh=_SC_MESH,
    compiler_params=pltpu.CompilerParams(needs_layout_passes=False),
    scratch_types=[
        pltpu.VMEM((TPW,), jnp.int32),
        pltpu.VMEM((TPW, DP), jnp.int32),
        pltpu.SemaphoreType.DMA,
    ],
)(_sc_gather_body)


def _combine_body(sh_ref, y0_ref, y1_ref, w1_ref, w2_ref, out_ref):
    out_ref[...] = (_unpack_rows(sh_ref[...])
                    + w1_ref[...] * _unpack_rows(y0_ref[...])
                    + w2_ref[...] * _unpack_rows(y1_ref[...]))


def _combine(shared, yg0, yg1, w1, w2):
    BTC = 512
    return pl.pallas_call(
        _combine_body,
        grid=(T // BTC,),
        in_specs=[
            pl.BlockSpec((BTC, DP), lambda i: (i, 0)),
            pl.BlockSpec((BTC, DP), lambda i: (i, 0)),
            pl.BlockSpec((BTC, DP), lambda i: (i, 0)),
            pl.BlockSpec((BTC, 1), lambda i: (i, 0)),
            pl.BlockSpec((BTC, 1), lambda i: (i, 0)),
        ],
        out_specs=pl.BlockSpec((BTC, D), lambda i: (i, 0)),
        out_shape=jax.ShapeDtypeStruct((T, D), jnp.float32),
    )(shared, yg0, yg1, w1, w2)


def kernel(hidden_states, gate_weight, e_score_correction_bias, Wg, Wu, Wd,
           Wg_s, Wu_s, Wd_s):
    orig_shape = hidden_states.shape
    x = hidden_states.reshape(-1, orig_shape[-1])
    bias2 = e_score_correction_bias.reshape(1, E)

    i1, i2, w1, w2, cnt3, xb, shp = _gate(x, gate_weight, bias2, Wg_s, Wu_s, Wd_s)
    ev = jnp.concatenate([i1, i2], axis=0).reshape(NA)
    cnt = cnt3.reshape(NW * 16)
    pos, be, xs = _sc_route(ev, cnt, xb)
    ys = _group_mlp(be, xs, Wg, Wu, Wd)
    yg0, yg1 = _sc_gather(pos, ys)
    out = _combine(shp, yg0, yg1, w1, w2)
    return out.reshape(orig_shape)


# speedup vs baseline: 1.1971x; 1.0145x over previous
import functools

import jax
import jax.numpy as jnp
from jax import lax
from jax.experimental import pallas as pl
from jax.experimental.pallas import tpu as pltpu
from jax.experimental.pallas import tpu_sc as plsc

E = 8
TOP_K = 2
T = 2048
D = 1024
FF = 512
NEG = -1.0e30

BT = 256
NA = T * TOP_K
PN = NA + E * BT
NB = PN // BT
NW = 32
APW = NA // NW
TPW = T // NW
DP = D // 2
MHI = -65536
RND = 32768


def _silu(v):
    return v / (1.0 + jnp.exp(-v))


def _pack_rows(v):
    b = jax.lax.bitcast_convert_type(v, jnp.int32)
    hi = (b[:, :DP] + RND) & MHI
    lo = jax.lax.shift_right_logical(b[:, DP:] + RND, 16)
    return hi | lo


def _unpack_rows(p):
    hi = jax.lax.bitcast_convert_type(p & MHI, jnp.float32)
    lo = jax.lax.bitcast_convert_type(jax.lax.shift_left(p, 16), jnp.float32)
    return jnp.concatenate([hi, lo], axis=1)


def _subcnt(idx_col):
    eq = idx_col == jax.lax.broadcasted_iota(jnp.int32, (idx_col.shape[0], 16), 1)
    return jnp.sum(jnp.where(eq, 1, 0), axis=0, keepdims=True)


def _gate_body(x_ref, gw_ref, bias_ref, wgs_ref, wus_ref, wds_ref,
               i1_ref, i2_ref, w1_ref, w2_ref, cnt_ref, xb_ref, sh_ref):
    logits = jax.lax.dot_general(x_ref[...], gw_ref[...], (((1,), (1,)), ((), ())),
                                 preferred_element_type=jnp.float32)
    logits = logits + bias_ref[...]
    iota = jax.lax.broadcasted_iota(jnp.int32, logits.shape, 1)
    m1 = jnp.max(logits, axis=1, keepdims=True)
    i1 = jnp.min(jnp.where(logits == m1, iota, E), axis=1, keepdims=True)
    masked = jnp.where(iota == i1, NEG, logits)
    m2 = jnp.max(masked, axis=1, keepdims=True)
    i2 = jnp.min(jnp.where(masked == m2, iota, E), axis=1, keepdims=True)
    e2 = jnp.exp(m2 - m1)
    w1 = 1.0 / (1.0 + e2)
    i1_ref[...] = i1
    i2_ref[...] = i2
    w1_ref[...] = w1
    w2_ref[...] = 1.0 - w1
    h = jnp.concatenate([
        _subcnt(i1[k * APW:(k + 1) * APW, :]) if s == 0
        else _subcnt(i2[k * APW:(k + 1) * APW, :])
        for k in range(4) for s in range(2)
    ], axis=0)
    cnt_ref[...] = h.reshape(4, 2, 16)
    xb_ref[...] = _pack_rows(x_ref[...])
    x = x_ref[...]
    gs = jax.lax.dot_general(x, wgs_ref[...], (((1,), (1,)), ((), ())),
                             preferred_element_type=jnp.float32)
    us = jax.lax.dot_general(x, wus_ref[...], (((1,), (1,)), ((), ())),
                             preferred_element_type=jnp.float32)
    hs = _silu(gs) * us
    sh = jax.lax.dot_general(hs, wds_ref[...], (((1,), (1,)), ((), ())),
                             preferred_element_type=jnp.float32)
    sh_ref[...] = _pack_rows(sh)


def _gate(x, gate_weight, bias2, Wg_s, Wu_s, Wd_s):
    BTG = 512
    return pl.pallas_call(
        _gate_body,
        grid=(T // BTG,),
        in_specs=[
            pl.BlockSpec((BTG, D), lambda i: (i, 0)),
            pl.BlockSpec((E, D), lambda i: (0, 0)),
            pl.BlockSpec((1, E), lambda i: (0, 0)),
            pl.BlockSpec(Wg_s.shape, lambda i: (0, 0)),
            pl.BlockSpec(Wu_s.shape, lambda i: (0, 0)),
            pl.BlockSpec(Wd_s.shape, lambda i: (0, 0)),
        ],
        out_specs=[
            pl.BlockSpec((BTG, 1), lambda i: (i, 0)),
            pl.BlockSpec((BTG, 1), lambda i: (i, 0)),
            pl.BlockSpec((BTG, 1), lambda i: (i, 0)),
            pl.BlockSpec((BTG, 1), lambda i: (i, 0)),
            pl.BlockSpec((4, 2, 16), lambda i: (i, 0, 0)),
            pl.BlockSpec((BTG, DP), lambda i: (i, 0)),
            pl.BlockSpec((BTG, DP), lambda i: (i, 0)),
        ],
        out_shape=[
            jax.ShapeDtypeStruct((T, 1), jnp.int32),
            jax.ShapeDtypeStruct((T, 1), jnp.int32),
            jax.ShapeDtypeStruct((T, 1), jnp.float32),
            jax.ShapeDtypeStruct((T, 1), jnp.float32),
            jax.ShapeDtypeStruct((T // APW, 2, 16), jnp.int32),
            jax.ShapeDtypeStruct((T, DP), jnp.int32),
            jax.ShapeDtypeStruct((T, DP), jnp.int32),
        ],
    )(x, gate_weight, bias2, Wg_s, Wu_s, Wd_s)


_SC_MESH = plsc.VectorSubcoreMesh(core_axis_name="c", subcore_axis_name="s",
                                  num_cores=2, num_subcores=16)


def _wid():
    return lax.axis_index("s") * 2 + lax.axis_index("c")


def _sc_route_body(ev_hbm, cnt_hbm, x_hbm, pos_hbm, be_hbm, xs_hbm,
                   ev_v, cnt_all_v, idx0_v, idx1_v, xrows_v, be_v, sem):
    wid = _wid()
    iota16 = lax.iota(jnp.int32, 16)
    pltpu.sync_copy(cnt_hbm, cnt_all_v)
    pltpu.sync_copy(ev_hbm.at[pl.ds(wid * APW, APW)], ev_v)

    tot = jnp.zeros((16,), jnp.int32)
    base_mine = jnp.zeros((16,), jnp.int32)
    for w in range(NW):
        row = cnt_all_v[pl.ds((w % 16) * 32 + (w // 16) * 16, 16)]
        tot = tot + row
        base_mine = base_mine + jnp.where(w < wid, row, 0)
    padded = (tot + (BT - 1)) & (-BT)
    ends = plsc.cumsum(padded)
    off = ends - padded
    base_vec = off + base_mine

    for c in range(APW // 16):
        ch = ev_v[pl.ds(c * 16, 16)]
        poschunk = jnp.zeros((16,), jnp.int32)
        for e in range(E):
            m = ch == e
            mi = jnp.where(m, 1, 0)
            pc = plsc.cumsum(mi)
            base_e = jnp.sum(jnp.where(iota16 == e, base_vec, 0))
            poschunk = jnp.where(m, base_e + pc - 1, poschunk)
            base_vec = base_vec + jnp.where(iota16 == e, jnp.sum(mi), 0)
        if c < (APW // 32):
            idx0_v[pl.ds(c * 16, 16)] = poschunk
        else:
            idx1_v[pl.ds(c * 16 - APW // 2, 16)] = poschunk
    pltpu.sync_copy(idx0_v, pos_hbm.at[pl.ds(wid * APW, APW // 2)])
    pltpu.sync_copy(idx1_v, pos_hbm.at[pl.ds(wid * APW + APW // 2, APW // 2)])

    tok0 = (wid % 16) * APW
    half = APW // 2
    for h, idx_v in ((0, idx0_v), (1, idx1_v)):
        pltpu.sync_copy(x_hbm.at[pl.ds(tok0 + h * half, half)], xrows_v)
        pltpu.async_copy(xrows_v, xs_hbm.at[idx_v], sem).wait()

    @pl.when(wid == 0)
    def _():
        bev0 = jnp.zeros((16,), jnp.int32)
        bev1 = jnp.zeros((16,), jnp.int32)
        for e in range(E):
            end_e = jnp.sum(jnp.where(iota16 == e, ends, 0))
            bev0 = bev0 + jnp.where(iota16 * BT >= end_e, 1, 0)
            bev1 = bev1 + jnp.where((iota16 + 16) * BT >= end_e, 1, 0)
        be_v[pl.ds(0, 16)] = bev0
        be_v[pl.ds(16, 16)] = bev1
        pltpu.sync_copy(be_v, be_hbm)


_sc_route = functools.partial(
    pl.kernel,
    out_type=[
        jax.ShapeDtypeStruct((NA,), jnp.int32),
        jax.ShapeDtypeStruct((32,), jnp.int32),
        jax.ShapeDtypeStruct((PN, DP), jnp.int32),
    ],
    mesh=_SC_MESH,
    compiler_params=pltpu.CompilerParams(needs_layout_passes=False),
    scratch_types=[
        pltpu.VMEM((APW,), jnp.int32),
        pltpu.VMEM((NW * 16,), jnp.int32),
        pltpu.VMEM((APW // 2,), jnp.int32),
        pltpu.VMEM((APW // 2,), jnp.int32),
        pltpu.VMEM((APW // 2, DP), jnp.int32),
        pltpu.VMEM((32,), jnp.int32),
        pltpu.SemaphoreType.DMA,
    ],
)(_sc_route_body)


def _group_mlp_body(be_ref, x_ref, wg_ref, wu_ref, wd_ref, y_ref):
    e = be_ref[pl.program_id(0)]

    @pl.when(e < E)
    def _():
        x = _unpack_rows(x_ref[...])
        g = jax.lax.dot_general(x, wg_ref[0], (((1,), (1,)), ((), ())),
                                preferred_element_type=jnp.float32)
        u = jax.lax.dot_general(x, wu_ref[0], (((1,), (1,)), ((), ())),
                                preferred_element_type=jnp.float32)
        h = _silu(g) * u
        y = jax.lax.dot_general(h, wd_ref[0], (((1,), (1,)), ((), ())),
                                preferred_element_type=jnp.float32)
        y_ref[...] = _pack_rows(y)


def _group_mlp(be, xs, Wg, Wu, Wd):
    def wmap(i, s):
        return (jnp.minimum(s[i], E - 1), 0, 0)

    grid_spec = pltpu.PrefetchScalarGridSpec(
        num_scalar_prefetch=1,
        grid=(NB,),
        in_specs=[
            pl.BlockSpec((BT, DP), lambda i, s: (i, 0)),
            pl.BlockSpec((1, FF, D), wmap),
            pl.BlockSpec((1, FF, D), wmap),
            pl.BlockSpec((1, D, FF), wmap),
        ],
        out_specs=pl.BlockSpec((BT, DP), lambda i, s: (i, 0)),
    )
    return pl.pallas_call(
        _group_mlp_body,
        grid_spec=grid_spec,
        out_shape=jax.ShapeDtypeStruct((PN, DP), jnp.int32),
    )(be, xs, Wg, Wu, Wd)


def _sc_gather_body(pos_hbm, ys_hbm, yg0_hbm, yg1_hbm, idx_v, rows_v, sem):
    wid = _wid()
    for s, out_hbm in ((0, yg0_hbm), (1, yg1_hbm)):
        pltpu.sync_copy(pos_hbm.at[pl.ds(s * T + wid * TPW, TPW)], idx_v)
        pltpu.async_copy(ys_hbm.at[idx_v], rows_v, sem).wait()
        pltpu.sync_copy(rows_v, out_hbm.at[pl.ds(wid * TPW, TPW)])


_sc_gather = functools.partial(
    pl.kernel,
    out_type=[
        jax.ShapeDtypeStruct((T, DP), jnp.int32),
        jax.ShapeDtypeStruct((T, DP), jnp.int32),
    ],
    mesh=_SC_MESH,
    compiler_params=pltpu.CompilerParams(needs_layout_passes=False),
    scratch_types=[
        pltpu.VMEM((TPW,), jnp.int32),
        pltpu.VMEM((TPW, DP), jnp.int32),
        pltpu.SemaphoreType.DMA,
    ],
)(_sc_gather_body)


def _combine_body(sh_ref, y0_ref, y1_ref, w1_ref, w2_ref, out_ref):
    out_ref[...] = (_unpack_rows(sh_ref[...])
                    + w1_ref[...] * _unpack_rows(y0_ref[...])
                    + w2_ref[...] * _unpack_rows(y1_ref[...]))


def _combine(shared, yg0, yg1, w1, w2):
    BTC = 1024
    return pl.pallas_call(
        _combine_body,
        grid=(T // BTC,),
        in_specs=[
            pl.BlockSpec((BTC, DP), lambda i: (i, 0)),
            pl.BlockSpec((BTC, DP), lambda i: (i, 0)),
            pl.BlockSpec((BTC, DP), lambda i: (i, 0)),
            pl.BlockSpec((BTC, 1), lambda i: (i, 0)),
            pl.BlockSpec((BTC, 1), lambda i: (i, 0)),
        ],
        out_specs=pl.BlockSpec((BTC, D), lambda i: (i, 0)),
        out_shape=jax.ShapeDtypeStruct((T, D), jnp.float32),
    )(shared, yg0, yg1, w1, w2)


def kernel(hidden_states, gate_weight, e_score_correction_bias, Wg, Wu, Wd,
           Wg_s, Wu_s, Wd_s):
    orig_shape = hidden_states.shape
    x = hidden_states.reshape(-1, orig_shape[-1])
    bias2 = e_score_correction_bias.reshape(1, E)

    i1, i2, w1, w2, cnt3, xb, shp = _gate(x, gate_weight, bias2, Wg_s, Wu_s, Wd_s)
    ev = jnp.concatenate([i1, i2], axis=0).reshape(NA)
    cnt = cnt3.reshape(NW * 16)
    pos, be, xs = _sc_route(ev, cnt, xb)
    ys = _group_mlp(be, xs, Wg, Wu, Wd)
    yg0, yg1 = _sc_gather(pos, ys)
    out = _combine(shp, yg0, yg1, w1, w2)
    return out.reshape(orig_shape)


# speedup vs baseline: 1.2309x; 1.0283x over previous
import functools

import jax
import jax.numpy as jnp
from jax import lax
from jax.experimental import pallas as pl
from jax.experimental.pallas import tpu as pltpu
from jax.experimental.pallas import tpu_sc as plsc

E = 8
TOP_K = 2
T = 2048
D = 1024
FF = 512
NEG = -1.0e30

BT = 256
NA = T * TOP_K
PN = NA + E * BT
NB = PN // BT
NW = 32
APW = NA // NW
TPW = T // NW
DP = D // 2
MHI = -65536
RND = 32768


def _silu(v):
    return v / (1.0 + jnp.exp(-v))


def _pack_rows(v):
    b = jax.lax.bitcast_convert_type(v, jnp.int32)
    hi = (b[:, :DP] + RND) & MHI
    lo = jax.lax.shift_right_logical(b[:, DP:] + RND, 16)
    return hi | lo


def _unpack_rows(p):
    hi = jax.lax.bitcast_convert_type(p & MHI, jnp.float32)
    lo = jax.lax.bitcast_convert_type(jax.lax.shift_left(p, 16), jnp.float32)
    return jnp.concatenate([hi, lo], axis=1)


def _subcnt(idx_col):
    eq = idx_col == jax.lax.broadcasted_iota(jnp.int32, (idx_col.shape[0], 16), 1)
    return jnp.sum(jnp.where(eq, 1, 0), axis=0, keepdims=True)


def _gate_body(x_ref, gw_ref, bias_ref, wgs_ref, wus_ref, wds_ref,
               i1_ref, i2_ref, w1_ref, w2_ref, cnt_ref, xb_ref, sh_ref):
    logits = jax.lax.dot_general(x_ref[...], gw_ref[...], (((1,), (1,)), ((), ())),
                                 preferred_element_type=jnp.float32)
    logits = logits + bias_ref[...]
    iota = jax.lax.broadcasted_iota(jnp.int32, logits.shape, 1)
    m1 = jnp.max(logits, axis=1, keepdims=True)
    i1 = jnp.min(jnp.where(logits == m1, iota, E), axis=1, keepdims=True)
    masked = jnp.where(iota == i1, NEG, logits)
    m2 = jnp.max(masked, axis=1, keepdims=True)
    i2 = jnp.min(jnp.where(masked == m2, iota, E), axis=1, keepdims=True)
    e2 = jnp.exp(m2 - m1)
    w1 = 1.0 / (1.0 + e2)
    i1_ref[...] = i1
    i2_ref[...] = i2
    w1_ref[...] = w1
    w2_ref[...] = 1.0 - w1
    h = jnp.concatenate([
        _subcnt(i1[k * APW:(k + 1) * APW, :]) if s == 0
        else _subcnt(i2[k * APW:(k + 1) * APW, :])
        for k in range(4) for s in range(2)
    ], axis=0)
    cnt_ref[...] = h.reshape(4, 2, 16)
    xb_ref[...] = _pack_rows(x_ref[...])
    x = x_ref[...]
    gs = jax.lax.dot_general(x, wgs_ref[...], (((1,), (1,)), ((), ())),
                             preferred_element_type=jnp.float32)
    us = jax.lax.dot_general(x, wus_ref[...], (((1,), (1,)), ((), ())),
                             preferred_element_type=jnp.float32)
    hs = _silu(gs) * us
    sh = jax.lax.dot_general(hs, wds_ref[...], (((1,), (1,)), ((), ())),
                             preferred_element_type=jnp.float32)
    sh_ref[...] = _pack_rows(sh)


def _gate(x, gate_weight, bias2, Wg_s, Wu_s, Wd_s):
    BTG = 512
    return pl.pallas_call(
        _gate_body,
        grid=(T // BTG,),
        in_specs=[
            pl.BlockSpec((BTG, D), lambda i: (i, 0)),
            pl.BlockSpec((E, D), lambda i: (0, 0)),
            pl.BlockSpec((1, E), lambda i: (0, 0)),
            pl.BlockSpec(Wg_s.shape, lambda i: (0, 0)),
            pl.BlockSpec(Wu_s.shape, lambda i: (0, 0)),
            pl.BlockSpec(Wd_s.shape, lambda i: (0, 0)),
        ],
        out_specs=[
            pl.BlockSpec((BTG, 1), lambda i: (i, 0)),
            pl.BlockSpec((BTG, 1), lambda i: (i, 0)),
            pl.BlockSpec((BTG, 1), lambda i: (i, 0)),
            pl.BlockSpec((BTG, 1), lambda i: (i, 0)),
            pl.BlockSpec((4, 2, 16), lambda i: (i, 0, 0)),
            pl.BlockSpec((BTG, DP), lambda i: (i, 0)),
            pl.BlockSpec((BTG, DP), lambda i: (i, 0)),
        ],
        out_shape=[
            jax.ShapeDtypeStruct((T, 1), jnp.int32),
            jax.ShapeDtypeStruct((T, 1), jnp.int32),
            jax.ShapeDtypeStruct((T, 1), jnp.float32),
            jax.ShapeDtypeStruct((T, 1), jnp.float32),
            jax.ShapeDtypeStruct((T // APW, 2, 16), jnp.int32),
            jax.ShapeDtypeStruct((T, DP), jnp.int32),
            jax.ShapeDtypeStruct((T, DP), jnp.int32),
        ],
    )(x, gate_weight, bias2, Wg_s, Wu_s, Wd_s)


_SC_MESH = plsc.VectorSubcoreMesh(core_axis_name="c", subcore_axis_name="s",
                                  num_cores=2, num_subcores=16)


def _wid():
    return lax.axis_index("s") * 2 + lax.axis_index("c")


def _sc_route_body(ev_hbm, cnt_hbm, x_hbm, pos_hbm, be_hbm, xs_hbm,
                   ev_v, cnt_all_v, idx0_v, idx1_v, xrows0_v, xrows1_v, be_v,
                   sem0, sem1):
    wid = _wid()
    iota16 = lax.iota(jnp.int32, 16)
    tok0 = (wid % 16) * APW
    half = APW // 2
    cx0 = pltpu.async_copy(x_hbm.at[pl.ds(tok0, half)], xrows0_v, sem0)
    cx1 = pltpu.async_copy(x_hbm.at[pl.ds(tok0 + half, half)], xrows1_v, sem1)
    pltpu.sync_copy(cnt_hbm, cnt_all_v)
    pltpu.sync_copy(ev_hbm.at[pl.ds(wid * APW, APW)], ev_v)

    tot = jnp.zeros((16,), jnp.int32)
    base_mine = jnp.zeros((16,), jnp.int32)
    for w in range(NW):
        row = cnt_all_v[pl.ds((w % 16) * 32 + (w // 16) * 16, 16)]
        tot = tot + row
        base_mine = base_mine + jnp.where(w < wid, row, 0)
    padded = (tot + (BT - 1)) & (-BT)
    ends = plsc.cumsum(padded)
    off = ends - padded
    base_vec = off + base_mine

    for c in range(APW // 16):
        ch = ev_v[pl.ds(c * 16, 16)]
        poschunk = jnp.zeros((16,), jnp.int32)
        for e in range(E):
            m = ch == e
            mi = jnp.where(m, 1, 0)
            pc = plsc.cumsum(mi)
            base_e = jnp.sum(jnp.where(iota16 == e, base_vec, 0))
            poschunk = jnp.where(m, base_e + pc - 1, poschunk)
            base_vec = base_vec + jnp.where(iota16 == e, jnp.sum(mi), 0)
        if c < (APW // 32):
            idx0_v[pl.ds(c * 16, 16)] = poschunk
        else:
            idx1_v[pl.ds(c * 16 - APW // 2, 16)] = poschunk
    pltpu.sync_copy(idx0_v, pos_hbm.at[pl.ds(wid * APW, APW // 2)])
    pltpu.sync_copy(idx1_v, pos_hbm.at[pl.ds(wid * APW + APW // 2, APW // 2)])

    cx0.wait()
    cs0 = pltpu.async_copy(xrows0_v, xs_hbm.at[idx0_v], sem0)
    cx1.wait()
    cs1 = pltpu.async_copy(xrows1_v, xs_hbm.at[idx1_v], sem1)
    cs0.wait()
    cs1.wait()

    @pl.when(wid == 0)
    def _():
        bev0 = jnp.zeros((16,), jnp.int32)
        bev1 = jnp.zeros((16,), jnp.int32)
        for e in range(E):
            end_e = jnp.sum(jnp.where(iota16 == e, ends, 0))
            bev0 = bev0 + jnp.where(iota16 * BT >= end_e, 1, 0)
            bev1 = bev1 + jnp.where((iota16 + 16) * BT >= end_e, 1, 0)
        be_v[pl.ds(0, 16)] = bev0
        be_v[pl.ds(16, 16)] = bev1
        pltpu.sync_copy(be_v, be_hbm)


_sc_route = functools.partial(
    pl.kernel,
    out_type=[
        jax.ShapeDtypeStruct((NA,), jnp.int32),
        jax.ShapeDtypeStruct((32,), jnp.int32),
        jax.ShapeDtypeStruct((PN, DP), jnp.int32),
    ],
    mesh=_SC_MESH,
    compiler_params=pltpu.CompilerParams(needs_layout_passes=False),
    scratch_types=[
        pltpu.VMEM((APW,), jnp.int32),
        pltpu.VMEM((NW * 16,), jnp.int32),
        pltpu.VMEM((APW // 2,), jnp.int32),
        pltpu.VMEM((APW // 2,), jnp.int32),
        pltpu.VMEM((APW // 2, DP), jnp.int32),
        pltpu.VMEM((APW // 2, DP), jnp.int32),
        pltpu.VMEM((32,), jnp.int32),
        pltpu.SemaphoreType.DMA,
        pltpu.SemaphoreType.DMA,
    ],
)(_sc_route_body)


def _group_mlp_body(be_ref, x_ref, wg_ref, wu_ref, wd_ref, y_ref):
    e = be_ref[pl.program_id(0)]

    @pl.when(e < E)
    def _():
        x = _unpack_rows(x_ref[...])
        g = jax.lax.dot_general(x, wg_ref[0], (((1,), (1,)), ((), ())),
                                preferred_element_type=jnp.float32)
        u = jax.lax.dot_general(x, wu_ref[0], (((1,), (1,)), ((), ())),
                                preferred_element_type=jnp.float32)
        h = _silu(g) * u
        y = jax.lax.dot_general(h, wd_ref[0], (((1,), (1,)), ((), ())),
                                preferred_element_type=jnp.float32)
        y_ref[...] = _pack_rows(y)


def _group_mlp(be, xs, Wg, Wu, Wd):
    def wmap(i, s):
        return (jnp.minimum(s[i], E - 1), 0, 0)

    grid_spec = pltpu.PrefetchScalarGridSpec(
        num_scalar_prefetch=1,
        grid=(NB,),
        in_specs=[
            pl.BlockSpec((BT, DP), lambda i, s: (i, 0)),
            pl.BlockSpec((1, FF, D), wmap),
            pl.BlockSpec((1, FF, D), wmap),
            pl.BlockSpec((1, D, FF), wmap),
        ],
        out_specs=pl.BlockSpec((BT, DP), lambda i, s: (i, 0)),
    )
    return pl.pallas_call(
        _group_mlp_body,
        grid_spec=grid_spec,
        out_shape=jax.ShapeDtypeStruct((PN, DP), jnp.int32),
    )(be, xs, Wg, Wu, Wd)


def _sc_gather_body(pos_hbm, ys_hbm, yg0_hbm, yg1_hbm, idx_v, rows_v, sem):
    wid = _wid()
    for s, out_hbm in ((0, yg0_hbm), (1, yg1_hbm)):
        pltpu.sync_copy(pos_hbm.at[pl.ds(s * T + wid * TPW, TPW)], idx_v)
        pltpu.async_copy(ys_hbm.at[idx_v], rows_v, sem).wait()
        pltpu.sync_copy(rows_v, out_hbm.at[pl.ds(wid * TPW, TPW)])


_sc_gather = functools.partial(
    pl.kernel,
    out_type=[
        jax.ShapeDtypeStruct((T, DP), jnp.int32),
        jax.ShapeDtypeStruct((T, DP), jnp.int32),
    ],
    mesh=_SC_MESH,
    compiler_params=pltpu.CompilerParams(needs_layout_passes=False),
    scratch_types=[
        pltpu.VMEM((TPW,), jnp.int32),
        pltpu.VMEM((TPW, DP), jnp.int32),
        pltpu.SemaphoreType.DMA,
    ],
)(_sc_gather_body)


def _combine_body(sh_ref, y0_ref, y1_ref, w1_ref, w2_ref, out_ref):
    out_ref[...] = (_unpack_rows(sh_ref[...])
                    + w1_ref[...] * _unpack_rows(y0_ref[...])
                    + w2_ref[...] * _unpack_rows(y1_ref[...]))


def _combine(shared, yg0, yg1, w1, w2):
    BTC = 1024
    return pl.pallas_call(
        _combine_body,
        grid=(T // BTC,),
        in_specs=[
            pl.BlockSpec((BTC, DP), lambda i: (i, 0)),
            pl.BlockSpec((BTC, DP), lambda i: (i, 0)),
            pl.BlockSpec((BTC, DP), lambda i: (i, 0)),
            pl.BlockSpec((BTC, 1), lambda i: (i, 0)),
            pl.BlockSpec((BTC, 1), lambda i: (i, 0)),
        ],
        out_specs=pl.BlockSpec((BTC, D), lambda i: (i, 0)),
        out_shape=jax.ShapeDtypeStruct((T, D), jnp.float32),
    )(shared, yg0, yg1, w1, w2)


def kernel(hidden_states, gate_weight, e_score_correction_bias, Wg, Wu, Wd,
           Wg_s, Wu_s, Wd_s):
    orig_shape = hidden_states.shape
    x = hidden_states.reshape(-1, orig_shape[-1])
    bias2 = e_score_correction_bias.reshape(1, E)

    i1, i2, w1, w2, cnt3, xb, shp = _gate(x, gate_weight, bias2, Wg_s, Wu_s, Wd_s)
    ev = jnp.concatenate([i1, i2], axis=0).reshape(NA)
    cnt = cnt3.reshape(NW * 16)
    pos, be, xs = _sc_route(ev, cnt, xb)
    ys = _group_mlp(be, xs, Wg, Wu, Wd)
    yg0, yg1 = _sc_gather(pos, ys)
    out = _combine(shp, yg0, yg1, w1, w2)
    return out.reshape(orig_shape)


# speedup vs baseline: 1.2465x; 1.0126x over previous
import functools

import jax
import jax.numpy as jnp
from jax import lax
from jax.experimental import pallas as pl
from jax.experimental.pallas import tpu as pltpu
from jax.experimental.pallas import tpu_sc as plsc

E = 8
TOP_K = 2
T = 2048
D = 1024
FF = 512
NEG = -1.0e30

BT = 256
NA = T * TOP_K
PN = NA + E * BT
NB = PN // BT
NW = 32
APW = NA // NW
TPW = T // NW
DP = D // 2
MHI = -65536
RND = 32768


def _silu(v):
    return v / (1.0 + jnp.exp(-v))


def _pack_rows(v):
    b = jax.lax.bitcast_convert_type(v, jnp.int32)
    hi = (b[:, :DP] + RND) & MHI
    lo = jax.lax.shift_right_logical(b[:, DP:] + RND, 16)
    return hi | lo


def _unpack_rows(p):
    hi = jax.lax.bitcast_convert_type(p & MHI, jnp.float32)
    lo = jax.lax.bitcast_convert_type(jax.lax.shift_left(p, 16), jnp.float32)
    return jnp.concatenate([hi, lo], axis=1)


def _subcnt(idx_col):
    eq = idx_col == jax.lax.broadcasted_iota(jnp.int32, (idx_col.shape[0], 16), 1)
    return jnp.sum(jnp.where(eq, 1, 0), axis=0, keepdims=True)


def _gate_body(x_ref, gw_ref, bias_ref, wgs_ref, wus_ref, wds_ref,
               i1_ref, i2_ref, w1_ref, w2_ref, cnt_ref, xb_ref, sh_ref):
    logits = jax.lax.dot_general(x_ref[...], gw_ref[...], (((1,), (1,)), ((), ())),
                                 preferred_element_type=jnp.float32)
    logits = logits + bias_ref[...]
    iota = jax.lax.broadcasted_iota(jnp.int32, logits.shape, 1)
    m1 = jnp.max(logits, axis=1, keepdims=True)
    i1 = jnp.min(jnp.where(logits == m1, iota, E), axis=1, keepdims=True)
    masked = jnp.where(iota == i1, NEG, logits)
    m2 = jnp.max(masked, axis=1, keepdims=True)
    i2 = jnp.min(jnp.where(masked == m2, iota, E), axis=1, keepdims=True)
    e2 = jnp.exp(m2 - m1)
    w1 = 1.0 / (1.0 + e2)
    i1_ref[...] = i1
    i2_ref[...] = i2
    w1_ref[...] = w1
    w2_ref[...] = 1.0 - w1
    h = jnp.concatenate([
        _subcnt(i1[k * APW:(k + 1) * APW, :]) if s == 0
        else _subcnt(i2[k * APW:(k + 1) * APW, :])
        for k in range(4) for s in range(2)
    ], axis=0)
    cnt_ref[...] = h.reshape(4, 2, 16)
    xb_ref[...] = _pack_rows(x_ref[...])
    x = x_ref[...]
    gs = jax.lax.dot_general(x, wgs_ref[...], (((1,), (1,)), ((), ())),
                             preferred_element_type=jnp.float32)
    us = jax.lax.dot_general(x, wus_ref[...], (((1,), (1,)), ((), ())),
                             preferred_element_type=jnp.float32)
    hs = _silu(gs) * us
    sh = jax.lax.dot_general(hs, wds_ref[...], (((1,), (1,)), ((), ())),
                             preferred_element_type=jnp.float32)
    sh_ref[...] = _pack_rows(sh)


def _gate(x, gate_weight, bias2, Wg_s, Wu_s, Wd_s):
    BTG = 512
    return pl.pallas_call(
        _gate_body,
        grid=(T // BTG,),
        in_specs=[
            pl.BlockSpec((BTG, D), lambda i: (i, 0)),
            pl.BlockSpec((E, D), lambda i: (0, 0)),
            pl.BlockSpec((1, E), lambda i: (0, 0)),
            pl.BlockSpec(Wg_s.shape, lambda i: (0, 0)),
            pl.BlockSpec(Wu_s.shape, lambda i: (0, 0)),
            pl.BlockSpec(Wd_s.shape, lambda i: (0, 0)),
        ],
        out_specs=[
            pl.BlockSpec((BTG, 1), lambda i: (i, 0)),
            pl.BlockSpec((BTG, 1), lambda i: (i, 0)),
            pl.BlockSpec((BTG, 1), lambda i: (i, 0)),
            pl.BlockSpec((BTG, 1), lambda i: (i, 0)),
            pl.BlockSpec((4, 2, 16), lambda i: (i, 0, 0)),
            pl.BlockSpec((BTG, DP), lambda i: (i, 0)),
            pl.BlockSpec((BTG, DP), lambda i: (i, 0)),
        ],
        out_shape=[
            jax.ShapeDtypeStruct((T, 1), jnp.int32),
            jax.ShapeDtypeStruct((T, 1), jnp.int32),
            jax.ShapeDtypeStruct((T, 1), jnp.float32),
            jax.ShapeDtypeStruct((T, 1), jnp.float32),
            jax.ShapeDtypeStruct((T // APW, 2, 16), jnp.int32),
            jax.ShapeDtypeStruct((T, DP), jnp.int32),
            jax.ShapeDtypeStruct((T, DP), jnp.int32),
        ],
    )(x, gate_weight, bias2, Wg_s, Wu_s, Wd_s)


_SC_MESH = plsc.VectorSubcoreMesh(core_axis_name="c", subcore_axis_name="s",
                                  num_cores=2, num_subcores=16)


def _wid():
    return lax.axis_index("s") * 2 + lax.axis_index("c")


def _sc_route_body(ev_hbm, cnt_hbm, x_hbm, pos_hbm, be_hbm, xs_hbm,
                   ev_v, cnt_all_v, idx0_v, idx1_v, xrows0_v, xrows1_v, be_v,
                   sem0, sem1):
    wid = _wid()
    iota16 = lax.iota(jnp.int32, 16)
    tok0 = (wid % 16) * APW
    half = APW // 2
    cx0 = pltpu.async_copy(x_hbm.at[pl.ds(tok0, half)], xrows0_v, sem0)
    cx1 = pltpu.async_copy(x_hbm.at[pl.ds(tok0 + half, half)], xrows1_v, sem1)
    pltpu.sync_copy(cnt_hbm, cnt_all_v)
    pltpu.sync_copy(ev_hbm.at[pl.ds(wid * APW, APW)], ev_v)

    tot = jnp.zeros((16,), jnp.int32)
    base_mine = jnp.zeros((16,), jnp.int32)
    for w in range(NW):
        row = cnt_all_v[pl.ds((w % 16) * 32 + (w // 16) * 16, 16)]
        tot = tot + row
        base_mine = base_mine + jnp.where(w < wid, row, 0)
    padded = (tot + (BT - 1)) & (-BT)
    ends = plsc.cumsum(padded)
    off = ends - padded
    base_vec = off + base_mine

    for c in range(APW // 16):
        ch = ev_v[pl.ds(c * 16, 16)]
        poschunk = jnp.zeros((16,), jnp.int32)
        for e in range(E):
            m = ch == e
            mi = jnp.where(m, 1, 0)
            pc = plsc.cumsum(mi)
            base_e = jnp.sum(jnp.where(iota16 == e, base_vec, 0))
            poschunk = jnp.where(m, base_e + pc - 1, poschunk)
            base_vec = base_vec + jnp.where(iota16 == e, jnp.sum(mi), 0)
        if c < (APW // 32):
            idx0_v[pl.ds(c * 16, 16)] = poschunk
        else:
            idx1_v[pl.ds(c * 16 - APW // 2, 16)] = poschunk
    pltpu.sync_copy(idx0_v, pos_hbm.at[pl.ds(wid * APW, APW // 2)])
    pltpu.sync_copy(idx1_v, pos_hbm.at[pl.ds(wid * APW + APW // 2, APW // 2)])

    cx0.wait()
    cs0 = pltpu.async_copy(xrows0_v, xs_hbm.at[idx0_v], sem0)
    cx1.wait()
    cs1 = pltpu.async_copy(xrows1_v, xs_hbm.at[idx1_v], sem1)
    cs0.wait()
    cs1.wait()

    @pl.when(wid == 0)
    def _():
        bev0 = jnp.zeros((16,), jnp.int32)
        bev1 = jnp.zeros((16,), jnp.int32)
        for e in range(E):
            end_e = jnp.sum(jnp.where(iota16 == e, ends, 0))
            bev0 = bev0 + jnp.where(iota16 * BT >= end_e, 1, 0)
            bev1 = bev1 + jnp.where((iota16 + 16) * BT >= end_e, 1, 0)
        be_v[pl.ds(0, 16)] = bev0
        be_v[pl.ds(16, 16)] = bev1
        pltpu.sync_copy(be_v, be_hbm)


_sc_route = functools.partial(
    pl.kernel,
    out_type=[
        jax.ShapeDtypeStruct((NA,), jnp.int32),
        jax.ShapeDtypeStruct((32,), jnp.int32),
        jax.ShapeDtypeStruct((PN, DP), jnp.int32),
    ],
    mesh=_SC_MESH,
    compiler_params=pltpu.CompilerParams(needs_layout_passes=False),
    scratch_types=[
        pltpu.VMEM((APW,), jnp.int32),
        pltpu.VMEM((NW * 16,), jnp.int32),
        pltpu.VMEM((APW // 2,), jnp.int32),
        pltpu.VMEM((APW // 2,), jnp.int32),
        pltpu.VMEM((APW // 2, DP), jnp.int32),
        pltpu.VMEM((APW // 2, DP), jnp.int32),
        pltpu.VMEM((32,), jnp.int32),
        pltpu.SemaphoreType.DMA,
        pltpu.SemaphoreType.DMA,
    ],
)(_sc_route_body)


def _group_mlp_body(be_ref, x_ref, wg_ref, wu_ref, wd_ref, y_ref):
    e = be_ref[pl.program_id(0)]

    @pl.when(e < E)
    def _():
        x = _unpack_rows(x_ref[...])
        g = jax.lax.dot_general(x, wg_ref[0], (((1,), (1,)), ((), ())),
                                preferred_element_type=jnp.float32)
        u = jax.lax.dot_general(x, wu_ref[0], (((1,), (1,)), ((), ())),
                                preferred_element_type=jnp.float32)
        h = _silu(g) * u
        y = jax.lax.dot_general(h, wd_ref[0], (((1,), (1,)), ((), ())),
                                preferred_element_type=jnp.float32)
        y_ref[...] = _pack_rows(y)


def _group_mlp(be, xs, Wg, Wu, Wd):
    def wmap(i, s):
        return (jnp.minimum(s[i], E - 1), 0, 0)

    grid_spec = pltpu.PrefetchScalarGridSpec(
        num_scalar_prefetch=1,
        grid=(NB,),
        in_specs=[
            pl.BlockSpec((BT, DP), lambda i, s: (i, 0)),
            pl.BlockSpec((1, FF, D), wmap),
            pl.BlockSpec((1, FF, D), wmap),
            pl.BlockSpec((1, D, FF), wmap),
        ],
        out_specs=pl.BlockSpec((BT, DP), lambda i, s: (i, 0)),
    )
    return pl.pallas_call(
        _group_mlp_body,
        grid_spec=grid_spec,
        out_shape=jax.ShapeDtypeStruct((PN, DP), jnp.int32),
    )(be, xs, Wg, Wu, Wd)


def _sc_gather_body(pos_hbm, ys_hbm, yg0_hbm, yg1_hbm,
                    idx0_v, idx1_v, rows0_v, rows1_v, sem0, sem1):
    wid = _wid()
    pltpu.sync_copy(pos_hbm.at[pl.ds(wid * TPW, TPW)], idx0_v)
    g0 = pltpu.async_copy(ys_hbm.at[idx0_v], rows0_v, sem0)
    pltpu.sync_copy(pos_hbm.at[pl.ds(T + wid * TPW, TPW)], idx1_v)
    g1 = pltpu.async_copy(ys_hbm.at[idx1_v], rows1_v, sem1)
    g0.wait()
    w0 = pltpu.async_copy(rows0_v, yg0_hbm.at[pl.ds(wid * TPW, TPW)], sem0)
    g1.wait()
    w1 = pltpu.async_copy(rows1_v, yg1_hbm.at[pl.ds(wid * TPW, TPW)], sem1)
    w0.wait()
    w1.wait()


_sc_gather = functools.partial(
    pl.kernel,
    out_type=[
        jax.ShapeDtypeStruct((T, DP), jnp.int32),
        jax.ShapeDtypeStruct((T, DP), jnp.int32),
    ],
    mesh=_SC_MESH,
    compiler_params=pltpu.CompilerParams(needs_layout_passes=False),
    scratch_types=[
        pltpu.VMEM((TPW,), jnp.int32),
        pltpu.VMEM((TPW,), jnp.int32),
        pltpu.VMEM((TPW, DP), jnp.int32),
        pltpu.VMEM((TPW, DP), jnp.int32),
        pltpu.SemaphoreType.DMA,
        pltpu.SemaphoreType.DMA,
    ],
)(_sc_gather_body)


def _combine_body(sh_ref, y0_ref, y1_ref, w1_ref, w2_ref, out_ref):
    out_ref[...] = (_unpack_rows(sh_ref[...])
                    + w1_ref[...] * _unpack_rows(y0_ref[...])
                    + w2_ref[...] * _unpack_rows(y1_ref[...]))


def _combine(shared, yg0, yg1, w1, w2):
    BTC = 1024
    return pl.pallas_call(
        _combine_body,
        grid=(T // BTC,),
        in_specs=[
            pl.BlockSpec((BTC, DP), lambda i: (i, 0)),
            pl.BlockSpec((BTC, DP), lambda i: (i, 0)),
            pl.BlockSpec((BTC, DP), lambda i: (i, 0)),
            pl.BlockSpec((BTC, 1), lambda i: (i, 0)),
            pl.BlockSpec((BTC, 1), lambda i: (i, 0)),
        ],
        out_specs=pl.BlockSpec((BTC, D), lambda i: (i, 0)),
        out_shape=jax.ShapeDtypeStruct((T, D), jnp.float32),
    )(shared, yg0, yg1, w1, w2)


def kernel(hidden_states, gate_weight, e_score_correction_bias, Wg, Wu, Wd,
           Wg_s, Wu_s, Wd_s):
    orig_shape = hidden_states.shape
    x = hidden_states.reshape(-1, orig_shape[-1])
    bias2 = e_score_correction_bias.reshape(1, E)

    i1, i2, w1, w2, cnt3, xb, shp = _gate(x, gate_weight, bias2, Wg_s, Wu_s, Wd_s)
    ev = jnp.concatenate([i1, i2], axis=0).reshape(NA)
    cnt = cnt3.reshape(NW * 16)
    pos, be, xs = _sc_route(ev, cnt, xb)
    ys = _group_mlp(be, xs, Wg, Wu, Wd)
    yg0, yg1 = _sc_gather(pos, ys)
    out = _combine(shp, yg0, yg1, w1, w2)
    return out.reshape(orig_shape)
